# Initial kernel scaffold; baseline (speedup 1.0000x reference)
import jax
import jax.numpy as jnp
from jax.experimental import pallas as pl


def kernel(sub, obj, rel, edge_index, etype, ts, entity_emb, sub_rel_emb, obj_rel_emb, conv_W, conv_b, obj_cls_W, obj_cls_b, sub_cls_W, sub_cls_b):
    raise NotImplementedError("write your pallas kernel here")


# speedup vs baseline: 32.1421x; 32.1421x over previous
import functools

import jax
import jax.numpy as jnp
from jax import lax
from jax.experimental import pallas as pl
from jax.experimental.pallas import tpu as pltpu
from jax.experimental.pallas import tpu_sc as plsc

NUME = 10000
DIM = 128
NUMR = 8
R = 2 * NUMR
B = 1024
E = 320000

NC = 2
NS = 16
LANES = 16
NW = NC * NS
CH = 128
NCH = -(-E // (NW * CH))
E_PAD = NW * CH * NCH
JUNK = 1792
T = R * NUME + JUNK
TROWS = T // 128
REAL_ROWS = (R * NUME) // 128
DEG_STRIPE = T // NS
NUME_PAD = 10240
ACC_STRIPE = NUME_PAD // NS
M_T = 400
NT = NUME // M_T
N_PAD = 10240
N_T = 512
BW = B // NW

@functools.cache
def _mesh():
    return plsc.VectorSubcoreMesh(core_axis_name="c", subcore_axis_name="s")


def _wid():
    return lax.axis_index("s") * NC + lax.axis_index("c")


def _sc_prep_body(src_h, dst_h, et_h, zdeg_h, idxs_out, idxd_out, deg_out,
                  src_v, dst_v, et_v, idxs_v, idxd_v, ones_v, dego_sh, degi_sh):
    cid = lax.axis_index("c")
    sid = lax.axis_index("s")
    wid = _wid()
    for k in range(CH // LANES):
        ones_v[pl.ds(k * LANES, LANES)] = jnp.ones((LANES,), jnp.float32)
    st = pl.ds(sid * DEG_STRIPE, DEG_STRIPE)
    pltpu.sync_copy(zdeg_h.at[st], dego_sh.at[st])
    pltpu.sync_copy(zdeg_h.at[st], degi_sh.at[st])
    plsc.subcore_barrier()

    def body(j, carry):
        base = (wid * NCH + j) * CH
        sl = pl.ds(base, CH)
        pltpu.sync_copy(src_h.at[sl], src_v)
        pltpu.sync_copy(dst_h.at[sl], dst_v)
        pltpu.sync_copy(et_h.at[sl], et_v)
        for k in range(CH // LANES):
            s = pl.ds(k * LANES, LANES)
            ev = et_v[s] * NUME
            idxs_v[s] = ev + src_v[s]
            idxd_v[s] = ev + dst_v[s]
        pltpu.sync_copy(idxs_v, idxs_out.at[sl])
        pltpu.sync_copy(idxd_v, idxd_out.at[sl])
        pltpu.sync_copy(ones_v, dego_sh.at[idxs_v], add=True)
        pltpu.sync_copy(ones_v, degi_sh.at[idxd_v], add=True)
        return carry

    lax.fori_loop(0, NCH, body, 0)
    plsc.subcore_barrier()
    off = cid * 2 * T + sid * DEG_STRIPE
    pltpu.sync_copy(dego_sh.at[st], deg_out.at[pl.ds(off, DEG_STRIPE)])
    pltpu.sync_copy(degi_sh.at[st], deg_out.at[pl.ds(off + T, DEG_STRIPE)])


@functools.cache
def _sc_prep():
    return pl.kernel(
    _sc_prep_body,
    out_type=(
        jax.ShapeDtypeStruct((E_PAD,), jnp.int32),
        jax.ShapeDtypeStruct((E_PAD,), jnp.int32),
        jax.ShapeDtypeStruct((NC * 2 * T,), jnp.float32),
    ),
    mesh=_mesh(),
    compiler_params=pltpu.CompilerParams(needs_layout_passes=False),
    scratch_types=[
        pltpu.VMEM((CH,), jnp.int32),
        pltpu.VMEM((CH,), jnp.int32),
        pltpu.VMEM((CH,), jnp.int32),
        pltpu.VMEM((CH,), jnp.int32),
        pltpu.VMEM((CH,), jnp.int32),
        pltpu.VMEM((CH,), jnp.float32),
        pltpu.VMEM_SHARED((T,), jnp.float32),
        pltpu.VMEM_SHARED((T,), jnp.float32),
    ],
)


def _tc_norm_body(deg_ref, no_ref, ni_ref):
    d = deg_ref[...]
    do = d[0, 0] + d[1, 0]
    di = d[0, 1] + d[1, 1]
    rowi = lax.broadcasted_iota(jnp.int32, (TROWS, 128), 0)
    mask = rowi < REAL_ROWS
    no_ref[...] = jnp.where(mask, lax.rsqrt(jnp.maximum(do, 1.0)), 0.0)
    ni_ref[...] = jnp.where(mask, lax.rsqrt(jnp.maximum(di, 1.0)), 0.0)


def _tc_norm(degs4):
    return pl.pallas_call(
        _tc_norm_body,
        out_shape=(
            jax.ShapeDtypeStruct((TROWS, 128), jnp.float32),
            jax.ShapeDtypeStruct((TROWS, 128), jnp.float32),
        ),
    )(degs4)


def _sc_coef_body(normo_h, normi_h, idxs_h, idxd_h, c_out, idxg_out,
                  idx_v, a_v, b_v, g_v, sem):
    wid = _wid()

    def body(j, carry):
        base = (wid * NCH + j) * CH
        sl = pl.ds(base, CH)
        pltpu.sync_copy(idxs_h.at[sl], idx_v)
        pltpu.async_copy(normo_h.at[idx_v], a_v, sem).wait()
        for k in range(CH // LANES):
            s = pl.ds(k * LANES, LANES)
            g_v[s] = lax.rem(idx_v[s], R * NUME)
        pltpu.sync_copy(g_v, idxg_out.at[sl])
        pltpu.sync_copy(idxd_h.at[sl], idx_v)
        pltpu.async_copy(normi_h.at[idx_v], b_v, sem).wait()
        for k in range(CH // LANES):
            s = pl.ds(k * LANES, LANES)
            a_v[s] = a_v[s] * b_v[s]
        pltpu.sync_copy(a_v, c_out.at[sl])
        return carry

    lax.fori_loop(0, NCH, body, 0)


@functools.cache
def _sc_coef():
    return pl.kernel(
    _sc_coef_body,
    out_type=(
        jax.ShapeDtypeStruct((E_PAD,), jnp.float32),
        jax.ShapeDtypeStruct((E_PAD,), jnp.int32),
    ),
    mesh=_mesh(),
    compiler_params=pltpu.CompilerParams(needs_layout_passes=False),
    scratch_types=[
        pltpu.VMEM((CH,), jnp.int32),
        pltpu.VMEM((CH,), jnp.float32),
        pltpu.VMEM((CH,), jnp.float32),
        pltpu.VMEM((CH,), jnp.int32),
        pltpu.SemaphoreType.DMA,
    ],
)


def _tc_g0_body(h_ref, w_ref, g_ref):
    g_ref[...] = jnp.dot(h_ref[...], w_ref[0],
                         preferred_element_type=jnp.float32)


def _tc_g0(h, w):
    return pl.pallas_call(
        _tc_g0_body,
        grid=(R, NT),
        in_specs=[
            pl.BlockSpec((M_T, DIM), lambda r, i: (i, 0)),
            pl.BlockSpec((1, DIM, DIM), lambda r, i: (r, 0, 0)),
        ],
        out_specs=pl.BlockSpec((M_T, DIM), lambda r, i: (r * NT + i, 0)),
        out_shape=jax.ShapeDtypeStruct((R * NUME, DIM), jnp.float32),
    )(h, w)


def _tc_g1_body(p0_ref, p1_ref, cb_ref, w_ref, g_ref):
    bm = jnp.sum(cb_ref[...], axis=0, keepdims=True) * (1.0 / R)
    h = (p0_ref[...] + p1_ref[...]) * (1.0 / R) + bm
    g_ref[...] = jnp.dot(h, w_ref[0], preferred_element_type=jnp.float32)


def _tc_g1(p0, p1, cb, w):
    return pl.pallas_call(
        _tc_g1_body,
        grid=(R, NT),
        in_specs=[
            pl.BlockSpec((M_T, DIM), lambda r, i: (i, 0)),
            pl.BlockSpec((M_T, DIM), lambda r, i: (i, 0)),
            pl.BlockSpec((R, DIM), lambda r, i: (0, 0)),
            pl.BlockSpec((1, DIM, DIM), lambda r, i: (r, 0, 0)),
        ],
        out_specs=pl.BlockSpec((M_T, DIM), lambda r, i: (r * NT + i, 0)),
        out_shape=jax.ShapeDtypeStruct((R * NUME, DIM), jnp.float32),
    )(p0, p1, cb, w)


def _sc_scatter_body(g_h, idxg_h, c_h, dst_h, zrow_h, part_out,
                     idx_v, dst_v, c_v, rows_v, accum_sh, sem):
    cid = lax.axis_index("c")
    sid = lax.axis_index("s")
    wid = _wid()
    st = pl.ds(sid * ACC_STRIPE, ACC_STRIPE)
    pltpu.sync_copy(zrow_h.at[st], accum_sh.at[st])
    plsc.subcore_barrier()
    col0 = lax.iota(jnp.int32, LANES)

    def body(j, carry):
        base = (wid * NCH + j) * CH
        sl = pl.ds(base, CH)
        pltpu.sync_copy(idxg_h.at[sl], idx_v)
        cp = pltpu.async_copy(g_h.at[idx_v], rows_v, sem)
        pltpu.sync_copy(c_h.at[sl], c_v)
        pltpu.sync_copy(dst_h.at[sl], dst_v)
        cp.wait()

        def scale(i, c2):
            ii = jnp.zeros((LANES,), jnp.int32) + i
            cb = plsc.load_gather(c_v, [ii])
            for k in range(DIM // LANES):
                idxs = [ii, col0 + k * LANES]
                v = plsc.load_gather(rows_v, idxs)
                plsc.store_scatter(rows_v, idxs, v * cb)
            return c2

        lax.fori_loop(0, CH, scale, 0)
        pltpu.sync_copy(rows_v, accum_sh.at[dst_v], add=True)
        return carry

    lax.fori_loop(0, NCH, body, 0)
    plsc.subcore_barrier()
    pltpu.sync_copy(accum_sh.at[st], part_out.at[cid, st])


@functools.cache
def _sc_scatter():
    return pl.kernel(
    _sc_scatter_body,
    out_type=jax.ShapeDtypeStruct((NC, NUME_PAD, DIM), jnp.float32),
    mesh=_mesh(),
    compiler_params=pltpu.CompilerParams(needs_layout_passes=False),
    scratch_types=[
        pltpu.VMEM((CH,), jnp.int32),
        pltpu.VMEM((CH,), jnp.int32),
        pltpu.VMEM((CH,), jnp.float32),
        pltpu.VMEM((CH, DIM), jnp.float32),
        pltpu.VMEM_SHARED((NUME_PAD, DIM), jnp.float32),
        pltpu.SemaphoreType.DMA,
    ],
)


def _tc_h_body(p0_ref, p1_ref, cb_ref, h_ref):
    bm = jnp.sum(cb_ref[...], axis=0, keepdims=True) * (1.0 / R)
    h_ref[...] = (p0_ref[...] + p1_ref[...]) * (1.0 / R) + bm


def _tc_h(p0, p1, cb):
    return pl.pallas_call(
        _tc_h_body,
        grid=(NT,),
        in_specs=[
            pl.BlockSpec((M_T, DIM), lambda i: (i, 0)),
            pl.BlockSpec((M_T, DIM), lambda i: (i, 0)),
            pl.BlockSpec((R, DIM), lambda i: (0, 0)),
        ],
        out_specs=pl.BlockSpec((M_T, DIM), lambda i: (i, 0)),
        out_shape=jax.ShapeDtypeStruct((NUME, DIM), jnp.float32),
    )(p0, p1, cb)


def _sc_gather_body(h_h, sub_h, obj_h, rel_h, sre_h, ore_h,
                    hs_out, ho_out, sr_out, or_out,
                    i_v, hrows_v, rrows_v, sem):
    wid = _wid()
    sl = pl.ds(wid * BW, BW)
    pltpu.sync_copy(sub_h.at[sl], i_v)
    pltpu.async_copy(h_h.at[i_v], hrows_v, sem).wait()
    pltpu.sync_copy(hrows_v, hs_out.at[sl])
    pltpu.sync_copy(obj_h.at[sl], i_v)
    pltpu.async_copy(h_h.at[i_v], hrows_v, sem).wait()
    pltpu.sync_copy(hrows_v, ho_out.at[sl])
    pltpu.sync_copy(rel_h.at[sl], i_v)
    pltpu.async_copy(sre_h.at[i_v], rrows_v, sem).wait()
    pltpu.sync_copy(rrows_v, sr_out.at[sl])
    pltpu.async_copy(ore_h.at[i_v], rrows_v, sem).wait()
    pltpu.sync_copy(rrows_v, or_out.at[sl])


@functools.cache
def _sc_gather():
    return pl.kernel(
    _sc_gather_body,
    out_type=(
        jax.ShapeDtypeStruct((B, DIM), jnp.float32),
        jax.ShapeDtypeStruct((B, DIM), jnp.float32),
        jax.ShapeDtypeStruct((B, DIM), jnp.float32),
        jax.ShapeDtypeStruct((B, DIM), jnp.float32),
    ),
    mesh=_mesh(),
    compiler_params=pltpu.CompilerParams(needs_layout_passes=False),
    scratch_types=[
        pltpu.VMEM((BW,), jnp.int32),
        pltpu.VMEM((BW, DIM), jnp.float32),
        pltpu.VMEM((BW, DIM), jnp.float32),
        pltpu.SemaphoreType.DMA,
    ],
)


def _tc_cls_body(ho_ref, or_ref, hs_ref, sr_ref,
                 swe_ref, swr_ref, sb_ref, owe_ref, owr_ref, ob_ref,
                 sp_ref, op_ref):
    sp_ref[...] = (jnp.dot(ho_ref[...], swe_ref[...],
                           preferred_element_type=jnp.float32)
                   + jnp.dot(or_ref[...], swr_ref[...],
                             preferred_element_type=jnp.float32)
                   + sb_ref[...])
    op_ref[...] = (jnp.dot(hs_ref[...], owe_ref[...],
                           preferred_element_type=jnp.float32)
                   + jnp.dot(sr_ref[...], owr_ref[...],
                             preferred_element_type=jnp.float32)
                   + ob_ref[...])


def _tc_cls(ho, orr, hs, sr, swe, swr, sb, owe, owr, ob):
    return pl.pallas_call(
        _tc_cls_body,
        grid=(N_PAD // N_T,),
        in_specs=[
            pl.BlockSpec((B, DIM), lambda n: (0, 0)),
            pl.BlockSpec((B, 32), lambda n: (0, 0)),
            pl.BlockSpec((B, DIM), lambda n: (0, 0)),
            pl.BlockSpec((B, 32), lambda n: (0, 0)),
            pl.BlockSpec((DIM, N_T), lambda n: (0, n)),
            pl.BlockSpec((32, N_T), lambda n: (0, n)),
            pl.BlockSpec((1, N_T), lambda n: (0, n)),
            pl.BlockSpec((DIM, N_T), lambda n: (0, n)),
            pl.BlockSpec((32, N_T), lambda n: (0, n)),
            pl.BlockSpec((1, N_T), lambda n: (0, n)),
        ],
        out_specs=(
            pl.BlockSpec((B, N_T), lambda n: (0, n)),
            pl.BlockSpec((B, N_T), lambda n: (0, n)),
        ),
        out_shape=(
            jax.ShapeDtypeStruct((B, N_PAD), jnp.float32),
            jax.ShapeDtypeStruct((B, N_PAD), jnp.float32),
        ),
    )(ho, orr, hs, sr, swe, swr, sb, owe, owr, ob)


def kernel(sub, obj, rel, edge_index, etype, ts, entity_emb, sub_rel_emb,
           obj_rel_emb, conv_W, conv_b, obj_cls_W, obj_cls_b, sub_cls_W,
           sub_cls_b):
    src = edge_index[0].astype(jnp.int32)
    dst = edge_index[1].astype(jnp.int32)
    et = etype.astype(jnp.int32)
    pad = E_PAD - E
    padv = jnp.arange(pad, dtype=jnp.int32) % JUNK
    src_p = jnp.concatenate([src, padv])
    dst_p = jnp.concatenate([dst, padv])
    et_p = jnp.concatenate([et, jnp.full((pad,), R, jnp.int32)])
    zdeg = jnp.zeros((T,), jnp.float32)
    zrow = jnp.zeros((NUME_PAD, DIM), jnp.float32)

    idx_s, idx_d, degs = _sc_prep()(src_p, dst_p, et_p, zdeg)
    normo, normi = _tc_norm(degs.reshape(NC, 2, TROWS, 128))
    c, idx_g = _sc_coef()(normo.reshape(T), normi.reshape(T), idx_s, idx_d)

    g0 = _tc_g0(entity_emb, conv_W[0])
    parts0 = _sc_scatter()(g0, idx_g, c, dst_p, zrow)
    g1 = _tc_g1(parts0[0, :NUME], parts0[1, :NUME], conv_b[0], conv_W[1])
    parts1 = _sc_scatter()(g1, idx_g, c, dst_p, zrow)
    h2 = _tc_h(parts1[0, :NUME], parts1[1, :NUME], conv_b[1])

    srp = jnp.pad(sub_rel_emb, ((0, 0), (0, DIM - 32)))
    orp = jnp.pad(obj_rel_emb, ((0, 0), (0, DIM - 32)))
    hs, ho, sr, orr = _sc_gather()(h2, sub.astype(jnp.int32),
                                   obj.astype(jnp.int32),
                                   rel.astype(jnp.int32), srp, orp)
    sr = sr[:, :32]
    orr = orr[:, :32]

    npad = N_PAD - NUME
    swe = jnp.pad(sub_cls_W[:DIM], ((0, 0), (0, npad)))
    swr = jnp.pad(sub_cls_W[DIM:], ((0, 0), (0, npad)))
    sb = jnp.pad(sub_cls_b, (0, npad)).reshape(1, N_PAD)
    owe = jnp.pad(obj_cls_W[:DIM], ((0, 0), (0, npad)))
    owr = jnp.pad(obj_cls_W[DIM:], ((0, 0), (0, npad)))
    ob = jnp.pad(obj_cls_b, (0, npad)).reshape(1, N_PAD)
    sp, op_ = _tc_cls(ho, orr, hs, sr, swe, swr, sb, owe, owr, ob)
    return (sp[:, :NUME], op_[:, :NUME])


# speedup vs baseline: 38.6579x; 1.2027x over previous
import functools

import jax
import jax.numpy as jnp
from jax import lax
from jax.experimental import pallas as pl
from jax.experimental.pallas import tpu as pltpu
from jax.experimental.pallas import tpu_sc as plsc

NUME = 10000
DIM = 128
NUMR = 8
R = 2 * NUMR
B = 1024
E = 320000

NC = 2
NS = 16
LANES = 16
NW = NC * NS
CH = 64
GRP = 8
NCH = 160
NGRP = NCH // GRP
CHP = 128
NCHP = 80
E_PAD = NW * CH * NCH
P = 10240
T = R * P
TROWS = T // 128
DEG_STRIPE = T // NS
ACC_STRIPE = P // NS
M_T = 640
NT = P // M_T
N_PAD = 10240
N_T = 512
BW = B // NW


@functools.cache
def _mesh():
    return plsc.VectorSubcoreMesh(core_axis_name="c", subcore_axis_name="s")


def _wid():
    return lax.axis_index("s") * NC + lax.axis_index("c")


def _sc_prep_body(src_h, dst_h, et_h, zdeg_h, idxs_out, idxd_out, deg_out,
                  src_v, dst_v, et_v, idxs_v, idxd_v, ones_v, dego_sh, degi_sh):
    cid = lax.axis_index("c")
    sid = lax.axis_index("s")
    wid = _wid()
    for k in range(CHP // LANES):
        ones_v[pl.ds(k * LANES, LANES)] = jnp.ones((LANES,), jnp.float32)
    st = pl.ds(sid * DEG_STRIPE, DEG_STRIPE)
    pltpu.sync_copy(zdeg_h.at[st], dego_sh.at[st])
    pltpu.sync_copy(zdeg_h.at[st], degi_sh.at[st])
    plsc.subcore_barrier()

    def body(j, carry):
        base = (wid * NCHP + j) * CHP
        sl = pl.ds(base, CHP)
        pltpu.sync_copy(src_h.at[sl], src_v)
        pltpu.sync_copy(dst_h.at[sl], dst_v)
        pltpu.sync_copy(et_h.at[sl], et_v)
        for k in range(CHP // LANES):
            s = pl.ds(k * LANES, LANES)
            ev = et_v[s] * P
            idxs_v[s] = ev + src_v[s]
            idxd_v[s] = ev + dst_v[s]
        pltpu.sync_copy(idxs_v, idxs_out.at[sl])
        pltpu.sync_copy(idxd_v, idxd_out.at[sl])
        pltpu.sync_copy(ones_v, dego_sh.at[idxs_v], add=True)
        pltpu.sync_copy(ones_v, degi_sh.at[idxd_v], add=True)
        return carry

    lax.fori_loop(0, NCHP, body, 0)
    plsc.subcore_barrier()
    off = cid * 2 * T + sid * DEG_STRIPE
    pltpu.sync_copy(dego_sh.at[st], deg_out.at[pl.ds(off, DEG_STRIPE)])
    pltpu.sync_copy(degi_sh.at[st], deg_out.at[pl.ds(off + T, DEG_STRIPE)])


@functools.cache
def _sc_prep():
    return pl.kernel(
        _sc_prep_body,
        out_type=(
            jax.ShapeDtypeStruct((E_PAD,), jnp.int32),
            jax.ShapeDtypeStruct((E_PAD,), jnp.int32),
            jax.ShapeDtypeStruct((NC * 2 * T,), jnp.float32),
        ),
        mesh=_mesh(),
        compiler_params=pltpu.CompilerParams(needs_layout_passes=False),
        scratch_types=[
            pltpu.VMEM((CHP,), jnp.int32),
            pltpu.VMEM((CHP,), jnp.int32),
            pltpu.VMEM((CHP,), jnp.int32),
            pltpu.VMEM((CHP,), jnp.int32),
            pltpu.VMEM((CHP,), jnp.int32),
            pltpu.VMEM((CHP,), jnp.float32),
            pltpu.VMEM_SHARED((T,), jnp.float32),
            pltpu.VMEM_SHARED((T,), jnp.float32),
        ],
    )


def _tc_norm_body(deg_ref, no_ref, ni_ref):
    d = deg_ref[...]
    do = d[0, 0] + d[1, 0]
    di = d[0, 1] + d[1, 1]
    rowi = lax.broadcasted_iota(jnp.int32, (TROWS, 128), 0)
    coli = lax.broadcasted_iota(jnp.int32, (TROWS, 128), 1)
    mask = lax.rem(rowi * 128 + coli, P) < NUME
    no_ref[...] = jnp.where(mask, lax.rsqrt(jnp.maximum(do, 1.0)), 0.0)
    ni_ref[...] = jnp.where(mask, lax.rsqrt(jnp.maximum(di, 1.0)), 0.0)


def _tc_norm(degs4):
    return pl.pallas_call(
        _tc_norm_body,
        out_shape=(
            jax.ShapeDtypeStruct((TROWS, 128), jnp.float32),
            jax.ShapeDtypeStruct((TROWS, 128), jnp.float32),
        ),
    )(degs4)


def _sc_coef_body(normo_h, normi_h, idxs_h, idxd_h, c_out,
                  idx_v, a_v, b_v, sem):
    wid = _wid()

    def body(j, carry):
        base = (wid * NCHP + j) * CHP
        sl = pl.ds(base, CHP)
        pltpu.sync_copy(idxs_h.at[sl], idx_v)
        pltpu.async_copy(normo_h.at[idx_v], a_v, sem).wait()
        pltpu.sync_copy(idxd_h.at[sl], idx_v)
        pltpu.async_copy(normi_h.at[idx_v], b_v, sem).wait()
        for k in range(CHP // LANES):
            s = pl.ds(k * LANES, LANES)
            a_v[s] = a_v[s] * b_v[s]
        pltpu.sync_copy(a_v, c_out.at[sl])
        return carry

    lax.fori_loop(0, NCHP, body, 0)


@functools.cache
def _sc_coef():
    return pl.kernel(
        _sc_coef_body,
        out_type=jax.ShapeDtypeStruct((E_PAD,), jnp.float32),
        mesh=_mesh(),
        compiler_params=pltpu.CompilerParams(needs_layout_passes=False),
        scratch_types=[
            pltpu.VMEM((CHP,), jnp.int32),
            pltpu.VMEM((CHP,), jnp.float32),
            pltpu.VMEM((CHP,), jnp.float32),
            pltpu.SemaphoreType.DMA,
        ],
    )


def _tc_g0_body(h_ref, w_ref, g_ref):
    g_ref[...] = jnp.dot(h_ref[...], w_ref[0],
                         preferred_element_type=jnp.float32)


def _tc_g0(h, w):
    return pl.pallas_call(
        _tc_g0_body,
        grid=(R, NT),
        in_specs=[
            pl.BlockSpec((M_T, DIM), lambda r, i: (i, 0)),
            pl.BlockSpec((1, DIM, DIM), lambda r, i: (r, 0, 0)),
        ],
        out_specs=pl.BlockSpec((M_T, DIM), lambda r, i: (r * NT + i, 0)),
        out_shape=jax.ShapeDtypeStruct((R * P, DIM), jnp.float32),
    )(h, w)


def _tc_g1_body(p0_ref, p1_ref, cb_ref, w_ref, g_ref):
    bm = jnp.sum(cb_ref[...], axis=0, keepdims=True) * (1.0 / R)
    h = (p0_ref[0] + p1_ref[0]) * (1.0 / R) + bm
    g_ref[...] = jnp.dot(h, w_ref[0], preferred_element_type=jnp.float32)


def _tc_g1(parts, cb, w):
    return pl.pallas_call(
        _tc_g1_body,
        grid=(R, NT),
        in_specs=[
            pl.BlockSpec((1, M_T, DIM), lambda r, i: (0, i, 0)),
            pl.BlockSpec((1, M_T, DIM), lambda r, i: (1, i, 0)),
            pl.BlockSpec((R, DIM), lambda r, i: (0, 0)),
            pl.BlockSpec((1, DIM, DIM), lambda r, i: (r, 0, 0)),
        ],
        out_specs=pl.BlockSpec((M_T, DIM), lambda r, i: (r * NT + i, 0)),
        out_shape=jax.ShapeDtypeStruct((R * P, DIM), jnp.float32),
    )(parts, parts, cb, w)


def _sc_scatter_body(g_h, idx_h, c_h, dst_h, zrow_h, part_out,
                     idx8, c8, dst8, dc0, dc1, rg0, rg1, rs0, rs1, accum_sh,
                     gs0, gs1, ss0, ss1):
    cid = lax.axis_index("c")
    sid = lax.axis_index("s")
    wid = _wid()
    st = pl.ds(sid * ACC_STRIPE, ACC_STRIPE)
    pltpu.sync_copy(zrow_h.at[st], accum_sh.at[st])
    plsc.subcore_barrier()
    dc = [dc0, dc1]
    rg = [rg0, rg1]
    rs = [rs0, rs1]
    gs = [gs0, gs1]
    ss = [ss0, ss1]
    col0 = lax.iota(jnp.int32, LANES)
    base0 = wid * NCH * CH

    def group(gi, carry):
        gbase = base0 + gi * GRP * CH
        gsl = pl.ds(gbase, GRP * CH)
        pltpu.sync_copy(idx_h.at[gsl], idx8)
        pltpu.sync_copy(c_h.at[gsl], c8)
        pltpu.sync_copy(dst_h.at[gsl], dst8)
        pltpu.async_copy(g_h.at[idx8.at[pl.ds(0, CH)]], rg[0], gs[0])
        pltpu.async_copy(g_h.at[idx8.at[pl.ds(CH, CH)]], rg[1], gs[1])
        for k in range(GRP):
            b = k % 2
            pltpu.make_async_copy(
                g_h.at[idx8.at[pl.ds(k * CH, CH)]], rg[b], gs[b]).wait()
            if k >= 2:
                pltpu.make_async_copy(rs[b], accum_sh.at[dc[b]], ss[b]).wait()
            for q in range(CH // LANES):
                dc[b][pl.ds(q * LANES, LANES)] = dst8[
                    pl.ds(k * CH + q * LANES, LANES)]
            kk = jnp.full((LANES,), k * CH, jnp.int32)

            def scale(i, c2, _b=b, _kk=kk):
                ii = jnp.zeros((LANES,), jnp.int32) + i
                cb = plsc.load_gather(c8, [_kk + ii])
                for q in range(DIM // LANES):
                    idxs = [ii, col0 + q * LANES]
                    v = plsc.load_gather(rg[_b], idxs)
                    plsc.store_scatter(rs[_b], idxs, v * cb)
                return c2

            lax.fori_loop(0, CH, scale, 0)
            if k + 2 < GRP:
                pltpu.async_copy(
                    g_h.at[idx8.at[pl.ds((k + 2) * CH, CH)]], rg[b], gs[b])
            pltpu.async_copy(rs[b], accum_sh.at[dc[b]], ss[b], add=True)
        pltpu.make_async_copy(rs[0], accum_sh.at[dc[0]], ss[0]).wait()
        pltpu.make_async_copy(rs[1], accum_sh.at[dc[1]], ss[1]).wait()
        return carry

    lax.fori_loop(0, NGRP, group, 0)
    plsc.subcore_barrier()
    pltpu.sync_copy(accum_sh.at[st], part_out.at[cid, st])


@functools.cache
def _sc_scatter():
    return pl.kernel(
        _sc_scatter_body,
        out_type=jax.ShapeDtypeStruct((NC, P, DIM), jnp.float32),
        mesh=_mesh(),
        compiler_params=pltpu.CompilerParams(needs_layout_passes=False),
        scratch_types=[
            pltpu.VMEM((GRP * CH,), jnp.int32),
            pltpu.VMEM((GRP * CH,), jnp.float32),
            pltpu.VMEM((GRP * CH,), jnp.int32),
            pltpu.VMEM((CH,), jnp.int32),
            pltpu.VMEM((CH,), jnp.int32),
            pltpu.VMEM((CH, DIM), jnp.float32),
            pltpu.VMEM((CH, DIM), jnp.float32),
            pltpu.VMEM((CH, DIM), jnp.float32),
            pltpu.VMEM((CH, DIM), jnp.float32),
            pltpu.VMEM_SHARED((P, DIM), jnp.float32),
            pltpu.SemaphoreType.DMA,
            pltpu.SemaphoreType.DMA,
            pltpu.SemaphoreType.DMA,
            pltpu.SemaphoreType.DMA,
        ],
    )


def _tc_h_body(p0_ref, p1_ref, cb_ref, h_ref):
    bm = jnp.sum(cb_ref[...], axis=0, keepdims=True) * (1.0 / R)
    h_ref[...] = (p0_ref[0] + p1_ref[0]) * (1.0 / R) + bm


def _tc_h(parts, cb):
    return pl.pallas_call(
        _tc_h_body,
        grid=(NT,),
        in_specs=[
            pl.BlockSpec((1, M_T, DIM), lambda i: (0, i, 0)),
            pl.BlockSpec((1, M_T, DIM), lambda i: (1, i, 0)),
            pl.BlockSpec((R, DIM), lambda i: (0, 0)),
        ],
        out_specs=pl.BlockSpec((M_T, DIM), lambda i: (i, 0)),
        out_shape=jax.ShapeDtypeStruct((P, DIM), jnp.float32),
    )(parts, parts, cb)


def _sc_gather_body(h_h, sub_h, obj_h, rel_h, sre_h, ore_h,
                    hs_out, ho_out, sr_out, or_out,
                    i_v, hrows_v, rrows_v, sem):
    wid = _wid()
    sl = pl.ds(wid * BW, BW)
    pltpu.sync_copy(sub_h.at[sl], i_v)
    pltpu.async_copy(h_h.at[i_v], hrows_v, sem).wait()
    pltpu.sync_copy(hrows_v, hs_out.at[sl])
    pltpu.sync_copy(obj_h.at[sl], i_v)
    pltpu.async_copy(h_h.at[i_v], hrows_v, sem).wait()
    pltpu.sync_copy(hrows_v, ho_out.at[sl])
    pltpu.sync_copy(rel_h.at[sl], i_v)
    pltpu.async_copy(sre_h.at[i_v], rrows_v, sem).wait()
    pltpu.sync_copy(rrows_v, sr_out.at[sl])
    pltpu.async_copy(ore_h.at[i_v], rrows_v, sem).wait()
    pltpu.sync_copy(rrows_v, or_out.at[sl])


@functools.cache
def _sc_gather():
    return pl.kernel(
        _sc_gather_body,
        out_type=(
            jax.ShapeDtypeStruct((B, DIM), jnp.float32),
            jax.ShapeDtypeStruct((B, DIM), jnp.float32),
            jax.ShapeDtypeStruct((B, DIM), jnp.float32),
            jax.ShapeDtypeStruct((B, DIM), jnp.float32),
        ),
        mesh=_mesh(),
        compiler_params=pltpu.CompilerParams(needs_layout_passes=False),
        scratch_types=[
            pltpu.VMEM((BW,), jnp.int32),
            pltpu.VMEM((BW, DIM), jnp.float32),
            pltpu.VMEM((BW, DIM), jnp.float32),
            pltpu.SemaphoreType.DMA,
        ],
    )


def _tc_cls_body(ho_ref, or_ref, hs_ref, sr_ref,
                 swe_ref, swr_ref, sb_ref, owe_ref, owr_ref, ob_ref,
                 sp_ref, op_ref):
    sp_ref[...] = (jnp.dot(ho_ref[...], swe_ref[...],
                           preferred_element_type=jnp.float32)
                   + jnp.dot(or_ref[...], swr_ref[...],
                             preferred_element_type=jnp.float32)
                   + sb_ref[...])
    op_ref[...] = (jnp.dot(hs_ref[...], owe_ref[...],
                           preferred_element_type=jnp.float32)
                   + jnp.dot(sr_ref[...], owr_ref[...],
                             preferred_element_type=jnp.float32)
                   + ob_ref[...])


def _tc_cls(ho, orr, hs, sr, swe, swr, sb, owe, owr, ob):
    return pl.pallas_call(
        _tc_cls_body,
        grid=(N_PAD // N_T,),
        in_specs=[
            pl.BlockSpec((B, DIM), lambda n: (0, 0)),
            pl.BlockSpec((B, 32), lambda n: (0, 0)),
            pl.BlockSpec((B, DIM), lambda n: (0, 0)),
            pl.BlockSpec((B, 32), lambda n: (0, 0)),
            pl.BlockSpec((DIM, N_T), lambda n: (0, n)),
            pl.BlockSpec((32, N_T), lambda n: (0, n)),
            pl.BlockSpec((1, N_T), lambda n: (0, n)),
            pl.BlockSpec((DIM, N_T), lambda n: (0, n)),
            pl.BlockSpec((32, N_T), lambda n: (0, n)),
            pl.BlockSpec((1, N_T), lambda n: (0, n)),
        ],
        out_specs=(
            pl.BlockSpec((B, N_T), lambda n: (0, n)),
            pl.BlockSpec((B, N_T), lambda n: (0, n)),
        ),
        out_shape=(
            jax.ShapeDtypeStruct((B, N_PAD), jnp.float32),
            jax.ShapeDtypeStruct((B, N_PAD), jnp.float32),
        ),
    )(ho, orr, hs, sr, swe, swr, sb, owe, owr, ob)


def kernel(sub, obj, rel, edge_index, etype, ts, entity_emb, sub_rel_emb,
           obj_rel_emb, conv_W, conv_b, obj_cls_W, obj_cls_b, sub_cls_W,
           sub_cls_b):
    src = edge_index[0].astype(jnp.int32)
    dst = edge_index[1].astype(jnp.int32)
    et = etype.astype(jnp.int32)
    pad = E_PAD - E
    padv = NUME + (jnp.arange(pad, dtype=jnp.int32) % (P - NUME))
    src_p = jnp.concatenate([src, padv])
    dst_p = jnp.concatenate([dst, padv])
    et_p = jnp.concatenate([et, jnp.zeros((pad,), jnp.int32)])
    zdeg = jnp.zeros((T,), jnp.float32)
    zrow = jnp.zeros((P, DIM), jnp.float32)

    idx_s, idx_d, degs = _sc_prep()(src_p, dst_p, et_p, zdeg)
    normo, normi = _tc_norm(degs.reshape(NC, 2, TROWS, 128))
    c = _sc_coef()(normo.reshape(T), normi.reshape(T), idx_s, idx_d)

    emb_p = jnp.pad(entity_emb, ((0, P - NUME), (0, 0)))

    g0 = _tc_g0(emb_p, conv_W[0])
    parts0 = _sc_scatter()(g0, idx_s, c, dst_p, zrow)
    g1 = _tc_g1(parts0, conv_b[0], conv_W[1])
    parts1 = _sc_scatter()(g1, idx_s, c, dst_p, zrow)
    h2 = _tc_h(parts1, conv_b[1])

    srp = jnp.pad(sub_rel_emb, ((0, 0), (0, DIM - 32)))
    orp = jnp.pad(obj_rel_emb, ((0, 0), (0, DIM - 32)))
    hs, ho, sr, orr = _sc_gather()(h2, sub.astype(jnp.int32),
                                   obj.astype(jnp.int32),
                                   rel.astype(jnp.int32), srp, orp)
    sr = sr[:, :32]
    orr = orr[:, :32]

    npad = N_PAD - NUME
    swe = jnp.pad(sub_cls_W[:DIM], ((0, 0), (0, npad)))
    swr = jnp.pad(sub_cls_W[DIM:], ((0, 0), (0, npad)))
    sb = jnp.pad(sub_cls_b, (0, npad)).reshape(1, N_PAD)
    owe = jnp.pad(obj_cls_W[:DIM], ((0, 0), (0, npad)))
    owr = jnp.pad(obj_cls_W[DIM:], ((0, 0), (0, npad)))
    ob = jnp.pad(obj_cls_b, (0, npad)).reshape(1, N_PAD)
    sp, op_ = _tc_cls(ho, orr, hs, sr, swe, swr, sb, owe, owr, ob)
    return (sp[:, :NUME], op_[:, :NUME])


# speedup vs baseline: 40.1759x; 1.0393x over previous
import functools

import jax
import jax.numpy as jnp
from jax import lax
from jax.experimental import pallas as pl
from jax.experimental.pallas import tpu as pltpu
from jax.experimental.pallas import tpu_sc as plsc

NUME = 10000
DIM = 128
NUMR = 8
R = 2 * NUMR
B = 1024
E = 320000

NC = 2
NS = 16
LANES = 16
NW = NC * NS
CH = 64
GRP = 8
NCH = 160
NGRP = NCH // GRP
CHP = 128
NCHP = 80
E_PAD = NW * CH * NCH
P = 10240
T = R * P
TROWS = T // 128
DEG_STRIPE = T // NS
ACC_STRIPE = P // NS
M_T = 640
NT = P // M_T
N_PAD = 10240
N_T = 512
BW = B // NW


@functools.cache
def _mesh():
    return plsc.VectorSubcoreMesh(core_axis_name="c", subcore_axis_name="s")


def _wid():
    return lax.axis_index("s") * NC + lax.axis_index("c")


def _sc_prep_body(src_h, dst_h, et_h, zdeg_h, idxs_out, idxd_out, deg_out,
                  src_v, dst_v, et_v, idxs_v, idxd_v, ones_v, dego_sh, degi_sh):
    cid = lax.axis_index("c")
    sid = lax.axis_index("s")
    wid = _wid()
    for k in range(CHP // LANES):
        ones_v[pl.ds(k * LANES, LANES)] = jnp.ones((LANES,), jnp.float32)
    st = pl.ds(sid * DEG_STRIPE, DEG_STRIPE)
    pltpu.sync_copy(zdeg_h.at[st], dego_sh.at[st])
    pltpu.sync_copy(zdeg_h.at[st], degi_sh.at[st])
    plsc.subcore_barrier()

    def body(j, carry):
        base = (wid * NCHP + j) * CHP
        sl = pl.ds(base, CHP)
        pltpu.sync_copy(src_h.at[sl], src_v)
        pltpu.sync_copy(dst_h.at[sl], dst_v)
        pltpu.sync_copy(et_h.at[sl], et_v)
        for k in range(CHP // LANES):
            s = pl.ds(k * LANES, LANES)
            ev = et_v[s] * P
            idxs_v[s] = ev + src_v[s]
            idxd_v[s] = ev + dst_v[s]
        pltpu.sync_copy(idxs_v, idxs_out.at[sl])
        pltpu.sync_copy(idxd_v, idxd_out.at[sl])
        pltpu.sync_copy(ones_v, dego_sh.at[idxs_v], add=True)
        pltpu.sync_copy(ones_v, degi_sh.at[idxd_v], add=True)
        return carry

    lax.fori_loop(0, NCHP, body, 0)
    plsc.subcore_barrier()
    off = cid * 2 * T + sid * DEG_STRIPE
    pltpu.sync_copy(dego_sh.at[st], deg_out.at[pl.ds(off, DEG_STRIPE)])
    pltpu.sync_copy(degi_sh.at[st], deg_out.at[pl.ds(off + T, DEG_STRIPE)])


@functools.cache
def _sc_prep():
    return pl.kernel(
        _sc_prep_body,
        out_type=(
            jax.ShapeDtypeStruct((E_PAD,), jnp.int32),
            jax.ShapeDtypeStruct((E_PAD,), jnp.int32),
            jax.ShapeDtypeStruct((NC * 2 * T,), jnp.float32),
        ),
        mesh=_mesh(),
        compiler_params=pltpu.CompilerParams(needs_layout_passes=False),
        scratch_types=[
            pltpu.VMEM((CHP,), jnp.int32),
            pltpu.VMEM((CHP,), jnp.int32),
            pltpu.VMEM((CHP,), jnp.int32),
            pltpu.VMEM((CHP,), jnp.int32),
            pltpu.VMEM((CHP,), jnp.int32),
            pltpu.VMEM((CHP,), jnp.float32),
            pltpu.VMEM_SHARED((T,), jnp.float32),
            pltpu.VMEM_SHARED((T,), jnp.float32),
        ],
    )


def _tc_norm_body(deg_ref, no_ref, ni_ref):
    d = deg_ref[...]
    do = d[0, 0] + d[1, 0]
    di = d[0, 1] + d[1, 1]
    rowi = lax.broadcasted_iota(jnp.int32, (TROWS, 128), 0)
    coli = lax.broadcasted_iota(jnp.int32, (TROWS, 128), 1)
    mask = lax.rem(rowi * 128 + coli, P) < NUME
    no_ref[...] = jnp.where(mask, lax.rsqrt(jnp.maximum(do, 1.0)), 0.0)
    ni_ref[...] = jnp.where(mask, lax.rsqrt(jnp.maximum(di, 1.0)), 0.0)


def _tc_norm(degs4):
    return pl.pallas_call(
        _tc_norm_body,
        out_shape=(
            jax.ShapeDtypeStruct((TROWS, 128), jnp.float32),
            jax.ShapeDtypeStruct((TROWS, 128), jnp.float32),
        ),
    )(degs4)


def _sc_coef_body(normo_h, normi_h, idxs_h, idxd_h, c_out,
                  idx_v, a_v, b_v, sem):
    wid = _wid()

    def body(j, carry):
        base = (wid * NCHP + j) * CHP
        sl = pl.ds(base, CHP)
        pltpu.sync_copy(idxs_h.at[sl], idx_v)
        pltpu.async_copy(normo_h.at[idx_v], a_v, sem).wait()
        pltpu.sync_copy(idxd_h.at[sl], idx_v)
        pltpu.async_copy(normi_h.at[idx_v], b_v, sem).wait()
        for k in range(CHP // LANES):
            s = pl.ds(k * LANES, LANES)
            a_v[s] = a_v[s] * b_v[s]
        pltpu.sync_copy(a_v, c_out.at[sl])
        return carry

    lax.fori_loop(0, NCHP, body, 0)


@functools.cache
def _sc_coef():
    return pl.kernel(
        _sc_coef_body,
        out_type=jax.ShapeDtypeStruct((E_PAD,), jnp.float32),
        mesh=_mesh(),
        compiler_params=pltpu.CompilerParams(needs_layout_passes=False),
        scratch_types=[
            pltpu.VMEM((CHP,), jnp.int32),
            pltpu.VMEM((CHP,), jnp.float32),
            pltpu.VMEM((CHP,), jnp.float32),
            pltpu.SemaphoreType.DMA,
        ],
    )


def _tc_g0_body(h_ref, w_ref, g_ref):
    g_ref[...] = jnp.dot(h_ref[...], w_ref[0],
                         preferred_element_type=jnp.float32)


def _tc_g0(h, w):
    return pl.pallas_call(
        _tc_g0_body,
        grid=(R, NT),
        in_specs=[
            pl.BlockSpec((M_T, DIM), lambda r, i: (i, 0)),
            pl.BlockSpec((1, DIM, DIM), lambda r, i: (r, 0, 0)),
        ],
        out_specs=pl.BlockSpec((M_T, DIM), lambda r, i: (r * NT + i, 0)),
        out_shape=jax.ShapeDtypeStruct((R * P, DIM), jnp.float32),
    )(h, w)


def _tc_g1_body(p0_ref, p1_ref, cb_ref, w_ref, g_ref):
    bm = jnp.sum(cb_ref[...], axis=0, keepdims=True) * (1.0 / R)
    h = (p0_ref[0] + p1_ref[0]) * (1.0 / R) + bm
    g_ref[...] = jnp.dot(h, w_ref[0], preferred_element_type=jnp.float32)


def _tc_g1(parts, cb, w):
    return pl.pallas_call(
        _tc_g1_body,
        grid=(R, NT),
        in_specs=[
            pl.BlockSpec((1, M_T, DIM), lambda r, i: (0, i, 0)),
            pl.BlockSpec((1, M_T, DIM), lambda r, i: (1, i, 0)),
            pl.BlockSpec((R, DIM), lambda r, i: (0, 0)),
            pl.BlockSpec((1, DIM, DIM), lambda r, i: (r, 0, 0)),
        ],
        out_specs=pl.BlockSpec((M_T, DIM), lambda r, i: (r * NT + i, 0)),
        out_shape=jax.ShapeDtypeStruct((R * P, DIM), jnp.float32),
    )(parts, parts, cb, w)


def _sc_scatter_body(g_h, idx_h, c_h, dst_h, zrow_h, part_out,
                     idx8, c8, dst8, dc0, dc1, rg0, rg1, rs0, rs1, accum_sh,
                     gs0, gs1, ss0, ss1):
    cid = lax.axis_index("c")
    sid = lax.axis_index("s")
    wid = _wid()
    st = pl.ds(sid * ACC_STRIPE, ACC_STRIPE)
    pltpu.sync_copy(zrow_h.at[st], accum_sh.at[st])
    plsc.subcore_barrier()
    dc = [dc0, dc1]
    rg = [rg0, rg1]
    rs = [rs0, rs1]
    gs = [gs0, gs1]
    ss = [ss0, ss1]
    col0 = lax.iota(jnp.int32, LANES)
    base0 = wid * NCH * CH

    def group(gi, carry):
        gbase = base0 + gi * GRP * CH
        gsl = pl.ds(gbase, GRP * CH)
        pltpu.sync_copy(idx_h.at[gsl], idx8)
        pltpu.sync_copy(c_h.at[gsl], c8)
        pltpu.sync_copy(dst_h.at[gsl], dst8)
        pltpu.async_copy(g_h.at[idx8.at[pl.ds(0, CH)]], rg[0], gs[0])
        pltpu.async_copy(g_h.at[idx8.at[pl.ds(CH, CH)]], rg[1], gs[1])
        for k in range(GRP):
            b = k % 2
            pltpu.make_async_copy(
                g_h.at[idx8.at[pl.ds(k * CH, CH)]], rg[b], gs[b]).wait()
            if k >= 2:
                pltpu.make_async_copy(rs[b], accum_sh.at[dc[b]], ss[b]).wait()
            for q in range(CH // LANES):
                dc[b][pl.ds(q * LANES, LANES)] = dst8[
                    pl.ds(k * CH + q * LANES, LANES)]
            kk = jnp.full((LANES,), k * CH, jnp.int32)

            def scale(i, c2, _b=b, _kk=kk):
                ii = jnp.zeros((LANES,), jnp.int32) + i
                cb = plsc.load_gather(c8, [_kk + ii])
                for q in range(DIM // LANES):
                    idxs = [ii, col0 + q * LANES]
                    v = plsc.load_gather(rg[_b], idxs)
                    plsc.store_scatter(rs[_b], idxs, v * cb)
                return c2

            lax.fori_loop(0, CH, scale, 0, unroll=4)
            if k + 2 < GRP:
                pltpu.async_copy(
                    g_h.at[idx8.at[pl.ds((k + 2) * CH, CH)]], rg[b], gs[b])
            pltpu.async_copy(rs[b], accum_sh.at[dc[b]], ss[b], add=True)
        pltpu.make_async_copy(rs[0], accum_sh.at[dc[0]], ss[0]).wait()
        pltpu.make_async_copy(rs[1], accum_sh.at[dc[1]], ss[1]).wait()
        return carry

    lax.fori_loop(0, NGRP, group, 0)
    plsc.subcore_barrier()
    pltpu.sync_copy(accum_sh.at[st], part_out.at[cid, st])


@functools.cache
def _sc_scatter():
    return pl.kernel(
        _sc_scatter_body,
        out_type=jax.ShapeDtypeStruct((NC, P, DIM), jnp.float32),
        mesh=_mesh(),
        compiler_params=pltpu.CompilerParams(needs_layout_passes=False),
        scratch_types=[
            pltpu.VMEM((GRP * CH,), jnp.int32),
            pltpu.VMEM((GRP * CH,), jnp.float32),
            pltpu.VMEM((GRP * CH,), jnp.int32),
            pltpu.VMEM((CH,), jnp.int32),
            pltpu.VMEM((CH,), jnp.int32),
            pltpu.VMEM((CH, DIM), jnp.float32),
            pltpu.VMEM((CH, DIM), jnp.float32),
            pltpu.VMEM((CH, DIM), jnp.float32),
            pltpu.VMEM((CH, DIM), jnp.float32),
            pltpu.VMEM_SHARED((P, DIM), jnp.float32),
            pltpu.SemaphoreType.DMA,
            pltpu.SemaphoreType.DMA,
            pltpu.SemaphoreType.DMA,
            pltpu.SemaphoreType.DMA,
        ],
    )


def _tc_h_body(p0_ref, p1_ref, cb_ref, h_ref):
    bm = jnp.sum(cb_ref[...], axis=0, keepdims=True) * (1.0 / R)
    h_ref[...] = (p0_ref[0] + p1_ref[0]) * (1.0 / R) + bm


def _tc_h(parts, cb):
    return pl.pallas_call(
        _tc_h_body,
        grid=(NT,),
        in_specs=[
            pl.BlockSpec((1, M_T, DIM), lambda i: (0, i, 0)),
            pl.BlockSpec((1, M_T, DIM), lambda i: (1, i, 0)),
            pl.BlockSpec((R, DIM), lambda i: (0, 0)),
        ],
        out_specs=pl.BlockSpec((M_T, DIM), lambda i: (i, 0)),
        out_shape=jax.ShapeDtypeStruct((P, DIM), jnp.float32),
    )(parts, parts, cb)


def _sc_gather_body(h_h, sub_h, obj_h, rel_h, sre_h, ore_h,
                    hs_out, ho_out, sr_out, or_out,
                    i_v, hrows_v, rrows_v, sem):
    wid = _wid()
    sl = pl.ds(wid * BW, BW)
    pltpu.sync_copy(sub_h.at[sl], i_v)
    pltpu.async_copy(h_h.at[i_v], hrows_v, sem).wait()
    pltpu.sync_copy(hrows_v, hs_out.at[sl])
    pltpu.sync_copy(obj_h.at[sl], i_v)
    pltpu.async_copy(h_h.at[i_v], hrows_v, sem).wait()
    pltpu.sync_copy(hrows_v, ho_out.at[sl])
    pltpu.sync_copy(rel_h.at[sl], i_v)
    pltpu.async_copy(sre_h.at[i_v], rrows_v, sem).wait()
    pltpu.sync_copy(rrows_v, sr_out.at[sl])
    pltpu.async_copy(ore_h.at[i_v], rrows_v, sem).wait()
    pltpu.sync_copy(rrows_v, or_out.at[sl])


@functools.cache
def _sc_gather():
    return pl.kernel(
        _sc_gather_body,
        out_type=(
            jax.ShapeDtypeStruct((B, DIM), jnp.float32),
            jax.ShapeDtypeStruct((B, DIM), jnp.float32),
            jax.ShapeDtypeStruct((B, DIM), jnp.float32),
            jax.ShapeDtypeStruct((B, DIM), jnp.float32),
        ),
        mesh=_mesh(),
        compiler_params=pltpu.CompilerParams(needs_layout_passes=False),
        scratch_types=[
            pltpu.VMEM((BW,), jnp.int32),
            pltpu.VMEM((BW, DIM), jnp.float32),
            pltpu.VMEM((BW, DIM), jnp.float32),
            pltpu.SemaphoreType.DMA,
        ],
    )


def _tc_cls_body(ho_ref, or_ref, hs_ref, sr_ref,
                 swe_ref, swr_ref, sb_ref, owe_ref, owr_ref, ob_ref,
                 sp_ref, op_ref):
    sp_ref[...] = (jnp.dot(ho_ref[...], swe_ref[...],
                           preferred_element_type=jnp.float32)
                   + jnp.dot(or_ref[...], swr_ref[...],
                             preferred_element_type=jnp.float32)
                   + sb_ref[...])
    op_ref[...] = (jnp.dot(hs_ref[...], owe_ref[...],
                           preferred_element_type=jnp.float32)
                   + jnp.dot(sr_ref[...], owr_ref[...],
                             preferred_element_type=jnp.float32)
                   + ob_ref[...])


def _tc_cls(ho, orr, hs, sr, swe, swr, sb, owe, owr, ob):
    return pl.pallas_call(
        _tc_cls_body,
        grid=(N_PAD // N_T,),
        in_specs=[
            pl.BlockSpec((B, DIM), lambda n: (0, 0)),
            pl.BlockSpec((B, 32), lambda n: (0, 0)),
            pl.BlockSpec((B, DIM), lambda n: (0, 0)),
            pl.BlockSpec((B, 32), lambda n: (0, 0)),
            pl.BlockSpec((DIM, N_T), lambda n: (0, n)),
            pl.BlockSpec((32, N_T), lambda n: (0, n)),
            pl.BlockSpec((1, N_T), lambda n: (0, n)),
            pl.BlockSpec((DIM, N_T), lambda n: (0, n)),
            pl.BlockSpec((32, N_T), lambda n: (0, n)),
            pl.BlockSpec((1, N_T), lambda n: (0, n)),
        ],
        out_specs=(
            pl.BlockSpec((B, N_T), lambda n: (0, n)),
            pl.BlockSpec((B, N_T), lambda n: (0, n)),
        ),
        out_shape=(
            jax.ShapeDtypeStruct((B, N_PAD), jnp.float32),
            jax.ShapeDtypeStruct((B, N_PAD), jnp.float32),
        ),
    )(ho, orr, hs, sr, swe, swr, sb, owe, owr, ob)


def kernel(sub, obj, rel, edge_index, etype, ts, entity_emb, sub_rel_emb,
           obj_rel_emb, conv_W, conv_b, obj_cls_W, obj_cls_b, sub_cls_W,
           sub_cls_b):
    src = edge_index[0].astype(jnp.int32)
    dst = edge_index[1].astype(jnp.int32)
    et = etype.astype(jnp.int32)
    pad = E_PAD - E
    padv = NUME + (jnp.arange(pad, dtype=jnp.int32) % (P - NUME))
    src_p = jnp.concatenate([src, padv])
    dst_p = jnp.concatenate([dst, padv])
    et_p = jnp.concatenate([et, jnp.zeros((pad,), jnp.int32)])
    zdeg = jnp.zeros((T,), jnp.float32)
    zrow = jnp.zeros((P, DIM), jnp.float32)

    idx_s, idx_d, degs = _sc_prep()(src_p, dst_p, et_p, zdeg)
    normo, normi = _tc_norm(degs.reshape(NC, 2, TROWS, 128))
    c = _sc_coef()(normo.reshape(T), normi.reshape(T), idx_s, idx_d)

    emb_p = jnp.pad(entity_emb, ((0, P - NUME), (0, 0)))

    g0 = _tc_g0(emb_p, conv_W[0])
    parts0 = _sc_scatter()(g0, idx_s, c, dst_p, zrow)
    g1 = _tc_g1(parts0, conv_b[0], conv_W[1])
    parts1 = _sc_scatter()(g1, idx_s, c, dst_p, zrow)
    h2 = _tc_h(parts1, conv_b[1])

    srp = jnp.pad(sub_rel_emb, ((0, 0), (0, DIM - 32)))
    orp = jnp.pad(obj_rel_emb, ((0, 0), (0, DIM - 32)))
    hs, ho, sr, orr = _sc_gather()(h2, sub.astype(jnp.int32),
                                   obj.astype(jnp.int32),
                                   rel.astype(jnp.int32), srp, orp)
    sr = sr[:, :32]
    orr = orr[:, :32]

    npad = N_PAD - NUME
    swe = jnp.pad(sub_cls_W[:DIM], ((0, 0), (0, npad)))
    swr = jnp.pad(sub_cls_W[DIM:], ((0, 0), (0, npad)))
    sb = jnp.pad(sub_cls_b, (0, npad)).reshape(1, N_PAD)
    owe = jnp.pad(obj_cls_W[:DIM], ((0, 0), (0, npad)))
    owr = jnp.pad(obj_cls_W[DIM:], ((0, 0), (0, npad)))
    ob = jnp.pad(obj_cls_b, (0, npad)).reshape(1, N_PAD)
    sp, op_ = _tc_cls(ho, orr, hs, sr, swe, swr, sb, owe, owr, ob)
    return (sp[:, :NUME], op_[:, :NUME])


# speedup vs baseline: 59.3386x; 1.4770x over previous
import functools

import jax
import jax.numpy as jnp
from jax import lax
from jax.experimental import pallas as pl
from jax.experimental.pallas import tpu as pltpu
from jax.experimental.pallas import tpu_sc as plsc

NUME = 10000
DIM = 128
NUMR = 8
R = 2 * NUMR
B = 1024
E = 320000

NC = 2
NS = 16
LANES = 16
NW = NC * NS
CH = 64
GRP = 8
NCH = 160
NGRP = NCH // GRP
CHP = 128
NCHP = 80
E_PAD = NW * CH * NCH
P = 10240
T = R * P
TROWS = T // 128
DEG_STRIPE = T // NS
ACC_STRIPE = P // NS
M_T = 640
NT = P // M_T
N_PAD = 10240
N_T = 512
BW = B // NW


@functools.cache
def _mesh():
    return plsc.VectorSubcoreMesh(core_axis_name="c", subcore_axis_name="s")


def _wid():
    return lax.axis_index("s") * NC + lax.axis_index("c")


def _sc_prep_body(src_h, dst_h, et_h, zdeg_h, idxs_out, idxd_out, deg_out,
                  src_v, dst_v, et_v, idxs_v, idxd_v, ones_v, dego_sh, degi_sh):
    cid = lax.axis_index("c")
    sid = lax.axis_index("s")
    wid = _wid()
    for k in range(CHP // LANES):
        ones_v[pl.ds(k * LANES, LANES)] = jnp.ones((LANES,), jnp.float32)
    st = pl.ds(sid * DEG_STRIPE, DEG_STRIPE)
    pltpu.sync_copy(zdeg_h.at[st], dego_sh.at[st])
    pltpu.sync_copy(zdeg_h.at[st], degi_sh.at[st])
    plsc.subcore_barrier()

    def body(j, carry):
        base = (wid * NCHP + j) * CHP
        sl = pl.ds(base, CHP)
        pltpu.sync_copy(src_h.at[sl], src_v)
        pltpu.sync_copy(dst_h.at[sl], dst_v)
        pltpu.sync_copy(et_h.at[sl], et_v)
        for k in range(CHP // LANES):
            s = pl.ds(k * LANES, LANES)
            ev = et_v[s] * P
            idxs_v[s] = ev + src_v[s]
            idxd_v[s] = ev + dst_v[s]
        pltpu.sync_copy(idxs_v, idxs_out.at[sl])
        pltpu.sync_copy(idxd_v, idxd_out.at[sl])
        pltpu.sync_copy(ones_v, dego_sh.at[idxs_v], add=True)
        pltpu.sync_copy(ones_v, degi_sh.at[idxd_v], add=True)
        return carry

    lax.fori_loop(0, NCHP, body, 0)
    plsc.subcore_barrier()
    off = cid * 2 * T + sid * DEG_STRIPE
    pltpu.sync_copy(dego_sh.at[st], deg_out.at[pl.ds(off, DEG_STRIPE)])
    pltpu.sync_copy(degi_sh.at[st], deg_out.at[pl.ds(off + T, DEG_STRIPE)])


@functools.cache
def _sc_prep():
    return pl.kernel(
        _sc_prep_body,
        out_type=(
            jax.ShapeDtypeStruct((E_PAD,), jnp.int32),
            jax.ShapeDtypeStruct((E_PAD,), jnp.int32),
            jax.ShapeDtypeStruct((NC * 2 * T,), jnp.float32),
        ),
        mesh=_mesh(),
        compiler_params=pltpu.CompilerParams(needs_layout_passes=False),
        scratch_types=[
            pltpu.VMEM((CHP,), jnp.int32),
            pltpu.VMEM((CHP,), jnp.int32),
            pltpu.VMEM((CHP,), jnp.int32),
            pltpu.VMEM((CHP,), jnp.int32),
            pltpu.VMEM((CHP,), jnp.int32),
            pltpu.VMEM((CHP,), jnp.float32),
            pltpu.VMEM_SHARED((T,), jnp.float32),
            pltpu.VMEM_SHARED((T,), jnp.float32),
        ],
    )


def _tc_norm_body(deg_ref, no_ref, ni_ref):
    d = deg_ref[...]
    do = d[0, 0] + d[1, 0]
    di = d[0, 1] + d[1, 1]
    rowi = lax.broadcasted_iota(jnp.int32, (TROWS, 128), 0)
    coli = lax.broadcasted_iota(jnp.int32, (TROWS, 128), 1)
    mask = lax.rem(rowi * 128 + coli, P) < NUME
    no_ref[...] = jnp.where(mask, lax.rsqrt(jnp.maximum(do, 1.0)), 0.0)
    ni_ref[...] = jnp.where(mask, lax.rsqrt(jnp.maximum(di, 1.0)), 0.0)


def _tc_norm(degs4):
    return pl.pallas_call(
        _tc_norm_body,
        out_shape=(
            jax.ShapeDtypeStruct((TROWS, 128), jnp.float32),
            jax.ShapeDtypeStruct((TROWS, 128), jnp.float32),
        ),
    )(degs4)


def _sc_coef_body(normo_h, normi_h, idxs_h, idxd_h, c_out,
                  idx_v, a_v, b_v, sem):
    wid = _wid()

    def body(j, carry):
        base = (wid * NCHP + j) * CHP
        sl = pl.ds(base, CHP)
        pltpu.sync_copy(idxs_h.at[sl], idx_v)
        pltpu.async_copy(normo_h.at[idx_v], a_v, sem).wait()
        pltpu.sync_copy(idxd_h.at[sl], idx_v)
        pltpu.async_copy(normi_h.at[idx_v], b_v, sem).wait()
        for k in range(CHP // LANES):
            s = pl.ds(k * LANES, LANES)
            a_v[s] = a_v[s] * b_v[s]
        pltpu.sync_copy(a_v, c_out.at[sl])
        return carry

    lax.fori_loop(0, NCHP, body, 0)


@functools.cache
def _sc_coef():
    return pl.kernel(
        _sc_coef_body,
        out_type=jax.ShapeDtypeStruct((E_PAD,), jnp.float32),
        mesh=_mesh(),
        compiler_params=pltpu.CompilerParams(needs_layout_passes=False),
        scratch_types=[
            pltpu.VMEM((CHP,), jnp.int32),
            pltpu.VMEM((CHP,), jnp.float32),
            pltpu.VMEM((CHP,), jnp.float32),
            pltpu.SemaphoreType.DMA,
        ],
    )


def _tc_g0_body(h_ref, w_ref, g_ref):
    g_ref[...] = jnp.dot(h_ref[...], w_ref[0],
                         preferred_element_type=jnp.float32)


def _tc_g0(h, w):
    return pl.pallas_call(
        _tc_g0_body,
        grid=(R, NT),
        in_specs=[
            pl.BlockSpec((M_T, DIM), lambda r, i: (i, 0)),
            pl.BlockSpec((1, DIM, DIM), lambda r, i: (r, 0, 0)),
        ],
        out_specs=pl.BlockSpec((M_T, DIM), lambda r, i: (r * NT + i, 0)),
        out_shape=jax.ShapeDtypeStruct((R * P, DIM), jnp.float32),
    )(h, w)


def _tc_g1_body(p0_ref, p1_ref, cb_ref, w_ref, g_ref):
    bm = jnp.sum(cb_ref[...], axis=0, keepdims=True) * (1.0 / R)
    h = (p0_ref[0] + p1_ref[0]) * (1.0 / R) + bm
    g_ref[...] = jnp.dot(h, w_ref[0], preferred_element_type=jnp.float32)


def _tc_g1(parts, cb, w):
    return pl.pallas_call(
        _tc_g1_body,
        grid=(R, NT),
        in_specs=[
            pl.BlockSpec((1, M_T, DIM), lambda r, i: (0, i, 0)),
            pl.BlockSpec((1, M_T, DIM), lambda r, i: (1, i, 0)),
            pl.BlockSpec((R, DIM), lambda r, i: (0, 0)),
            pl.BlockSpec((1, DIM, DIM), lambda r, i: (r, 0, 0)),
        ],
        out_specs=pl.BlockSpec((M_T, DIM), lambda r, i: (r * NT + i, 0)),
        out_shape=jax.ShapeDtypeStruct((R * P, DIM), jnp.float32),
    )(parts, parts, cb, w)


def _sc_scatter_body(g_h, idx_h, c_h, dst_h, zrow_h, part_out,
                     idx8, c8, dst8, dc0, dc1, rg0, rg1, rs0, rs1, accum_sh,
                     gs0, gs1, ss0, ss1):
    cid = lax.axis_index("c")
    sid = lax.axis_index("s")
    wid = _wid()
    st = pl.ds(sid * ACC_STRIPE, ACC_STRIPE)
    pltpu.sync_copy(zrow_h.at[st], accum_sh.at[st])
    plsc.subcore_barrier()
    dc = [dc0, dc1]
    rg = [rg0, rg1]
    rs = [rs0, rs1]
    gs = [gs0, gs1]
    ss = [ss0, ss1]
    col0 = lax.iota(jnp.int32, LANES)
    base0 = wid * NCH * CH

    def group(gi, carry):
        gbase = base0 + gi * GRP * CH
        gsl = pl.ds(gbase, GRP * CH)
        pltpu.sync_copy(idx_h.at[gsl], idx8)
        pltpu.sync_copy(c_h.at[gsl], c8)
        pltpu.sync_copy(dst_h.at[gsl], dst8)
        pltpu.async_copy(g_h.at[idx8.at[pl.ds(0, CH)]], rg[0], gs[0])
        pltpu.async_copy(g_h.at[idx8.at[pl.ds(CH, CH)]], rg[1], gs[1])
        for k in range(GRP):
            b = k % 2
            pltpu.make_async_copy(
                g_h.at[idx8.at[pl.ds(k * CH, CH)]], rg[b], gs[b]).wait()
            if k >= 2:
                pltpu.make_async_copy(rs[b], accum_sh.at[dc[b]], ss[b]).wait()
            for q in range(CH // LANES):
                dc[b][pl.ds(q * LANES, LANES)] = dst8[
                    pl.ds(k * CH + q * LANES, LANES)]
            kk = jnp.full((LANES,), k * CH, jnp.int32)

            @plsc.parallel_loop(0, CH, unroll=4)
            def scale(i, _b=b, _kk=kk):
                ii = jnp.zeros((LANES,), jnp.int32) + i
                cb = plsc.load_gather(c8, [_kk + ii])
                for q in range(DIM // LANES):
                    idxs = [ii, col0 + q * LANES]
                    v = plsc.load_gather(rg[_b], idxs)
                    plsc.store_scatter(rs[_b], idxs, v * cb)
            if k + 2 < GRP:
                pltpu.async_copy(
                    g_h.at[idx8.at[pl.ds((k + 2) * CH, CH)]], rg[b], gs[b])
            pltpu.async_copy(rs[b], accum_sh.at[dc[b]], ss[b], add=True)
        pltpu.make_async_copy(rs[0], accum_sh.at[dc[0]], ss[0]).wait()
        pltpu.make_async_copy(rs[1], accum_sh.at[dc[1]], ss[1]).wait()
        return carry

    lax.fori_loop(0, NGRP, group, 0)
    plsc.subcore_barrier()
    pltpu.sync_copy(accum_sh.at[st], part_out.at[cid, st])


@functools.cache
def _sc_scatter():
    return pl.kernel(
        _sc_scatter_body,
        out_type=jax.ShapeDtypeStruct((NC, P, DIM), jnp.float32),
        mesh=_mesh(),
        compiler_params=pltpu.CompilerParams(needs_layout_passes=False),
        scratch_types=[
            pltpu.VMEM((GRP * CH,), jnp.int32),
            pltpu.VMEM((GRP * CH,), jnp.float32),
            pltpu.VMEM((GRP * CH,), jnp.int32),
            pltpu.VMEM((CH,), jnp.int32),
            pltpu.VMEM((CH,), jnp.int32),
            pltpu.VMEM((CH, DIM), jnp.float32),
            pltpu.VMEM((CH, DIM), jnp.float32),
            pltpu.VMEM((CH, DIM), jnp.float32),
            pltpu.VMEM((CH, DIM), jnp.float32),
            pltpu.VMEM_SHARED((P, DIM), jnp.float32),
            pltpu.SemaphoreType.DMA,
            pltpu.SemaphoreType.DMA,
            pltpu.SemaphoreType.DMA,
            pltpu.SemaphoreType.DMA,
        ],
    )


def _tc_h_body(p0_ref, p1_ref, cb_ref, h_ref):
    bm = jnp.sum(cb_ref[...], axis=0, keepdims=True) * (1.0 / R)
    h_ref[...] = (p0_ref[0] + p1_ref[0]) * (1.0 / R) + bm


def _tc_h(parts, cb):
    return pl.pallas_call(
        _tc_h_body,
        grid=(NT,),
        in_specs=[
            pl.BlockSpec((1, M_T, DIM), lambda i: (0, i, 0)),
            pl.BlockSpec((1, M_T, DIM), lambda i: (1, i, 0)),
            pl.BlockSpec((R, DIM), lambda i: (0, 0)),
        ],
        out_specs=pl.BlockSpec((M_T, DIM), lambda i: (i, 0)),
        out_shape=jax.ShapeDtypeStruct((P, DIM), jnp.float32),
    )(parts, parts, cb)


def _sc_gather_body(h_h, sub_h, obj_h, rel_h, sre_h, ore_h,
                    hs_out, ho_out, sr_out, or_out,
                    i_v, hrows_v, rrows_v, sem):
    wid = _wid()
    sl = pl.ds(wid * BW, BW)
    pltpu.sync_copy(sub_h.at[sl], i_v)
    pltpu.async_copy(h_h.at[i_v], hrows_v, sem).wait()
    pltpu.sync_copy(hrows_v, hs_out.at[sl])
    pltpu.sync_copy(obj_h.at[sl], i_v)
    pltpu.async_copy(h_h.at[i_v], hrows_v, sem).wait()
    pltpu.sync_copy(hrows_v, ho_out.at[sl])
    pltpu.sync_copy(rel_h.at[sl], i_v)
    pltpu.async_copy(sre_h.at[i_v], rrows_v, sem).wait()
    pltpu.sync_copy(rrows_v, sr_out.at[sl])
    pltpu.async_copy(ore_h.at[i_v], rrows_v, sem).wait()
    pltpu.sync_copy(rrows_v, or_out.at[sl])


@functools.cache
def _sc_gather():
    return pl.kernel(
        _sc_gather_body,
        out_type=(
            jax.ShapeDtypeStruct((B, DIM), jnp.float32),
            jax.ShapeDtypeStruct((B, DIM), jnp.float32),
            jax.ShapeDtypeStruct((B, DIM), jnp.float32),
            jax.ShapeDtypeStruct((B, DIM), jnp.float32),
        ),
        mesh=_mesh(),
        compiler_params=pltpu.CompilerParams(needs_layout_passes=False),
        scratch_types=[
            pltpu.VMEM((BW,), jnp.int32),
            pltpu.VMEM((BW, DIM), jnp.float32),
            pltpu.VMEM((BW, DIM), jnp.float32),
            pltpu.SemaphoreType.DMA,
        ],
    )


def _tc_cls_body(ho_ref, or_ref, hs_ref, sr_ref,
                 swe_ref, swr_ref, sb_ref, owe_ref, owr_ref, ob_ref,
                 sp_ref, op_ref):
    sp_ref[...] = (jnp.dot(ho_ref[...], swe_ref[...],
                           preferred_element_type=jnp.float32)
                   + jnp.dot(or_ref[...], swr_ref[...],
                             preferred_element_type=jnp.float32)
                   + sb_ref[...])
    op_ref[...] = (jnp.dot(hs_ref[...], owe_ref[...],
                           preferred_element_type=jnp.float32)
                   + jnp.dot(sr_ref[...], owr_ref[...],
                             preferred_element_type=jnp.float32)
                   + ob_ref[...])


def _tc_cls(ho, orr, hs, sr, swe, swr, sb, owe, owr, ob):
    return pl.pallas_call(
        _tc_cls_body,
        grid=(N_PAD // N_T,),
        in_specs=[
            pl.BlockSpec((B, DIM), lambda n: (0, 0)),
            pl.BlockSpec((B, 32), lambda n: (0, 0)),
            pl.BlockSpec((B, DIM), lambda n: (0, 0)),
            pl.BlockSpec((B, 32), lambda n: (0, 0)),
            pl.BlockSpec((DIM, N_T), lambda n: (0, n)),
            pl.BlockSpec((32, N_T), lambda n: (0, n)),
            pl.BlockSpec((1, N_T), lambda n: (0, n)),
            pl.BlockSpec((DIM, N_T), lambda n: (0, n)),
            pl.BlockSpec((32, N_T), lambda n: (0, n)),
            pl.BlockSpec((1, N_T), lambda n: (0, n)),
        ],
        out_specs=(
            pl.BlockSpec((B, N_T), lambda n: (0, n)),
            pl.BlockSpec((B, N_T), lambda n: (0, n)),
        ),
        out_shape=(
            jax.ShapeDtypeStruct((B, N_PAD), jnp.float32),
            jax.ShapeDtypeStruct((B, N_PAD), jnp.float32),
        ),
    )(ho, orr, hs, sr, swe, swr, sb, owe, owr, ob)


def kernel(sub, obj, rel, edge_index, etype, ts, entity_emb, sub_rel_emb,
           obj_rel_emb, conv_W, conv_b, obj_cls_W, obj_cls_b, sub_cls_W,
           sub_cls_b):
    src = edge_index[0].astype(jnp.int32)
    dst = edge_index[1].astype(jnp.int32)
    et = etype.astype(jnp.int32)
    pad = E_PAD - E
    padv = NUME + (jnp.arange(pad, dtype=jnp.int32) % (P - NUME))
    src_p = jnp.concatenate([src, padv])
    dst_p = jnp.concatenate([dst, padv])
    et_p = jnp.concatenate([et, jnp.zeros((pad,), jnp.int32)])
    zdeg = jnp.zeros((T,), jnp.float32)
    zrow = jnp.zeros((P, DIM), jnp.float32)

    idx_s, idx_d, degs = _sc_prep()(src_p, dst_p, et_p, zdeg)
    normo, normi = _tc_norm(degs.reshape(NC, 2, TROWS, 128))
    c = _sc_coef()(normo.reshape(T), normi.reshape(T), idx_s, idx_d)

    emb_p = jnp.pad(entity_emb, ((0, P - NUME), (0, 0)))

    g0 = _tc_g0(emb_p, conv_W[0])
    parts0 = _sc_scatter()(g0, idx_s, c, dst_p, zrow)
    g1 = _tc_g1(parts0, conv_b[0], conv_W[1])
    parts1 = _sc_scatter()(g1, idx_s, c, dst_p, zrow)
    h2 = _tc_h(parts1, conv_b[1])

    srp = jnp.pad(sub_rel_emb, ((0, 0), (0, DIM - 32)))
    orp = jnp.pad(obj_rel_emb, ((0, 0), (0, DIM - 32)))
    hs, ho, sr, orr = _sc_gather()(h2, sub.astype(jnp.int32),
                                   obj.astype(jnp.int32),
                                   rel.astype(jnp.int32), srp, orp)
    sr = sr[:, :32]
    orr = orr[:, :32]

    npad = N_PAD - NUME
    swe = jnp.pad(sub_cls_W[:DIM], ((0, 0), (0, npad)))
    swr = jnp.pad(sub_cls_W[DIM:], ((0, 0), (0, npad)))
    sb = jnp.pad(sub_cls_b, (0, npad)).reshape(1, N_PAD)
    owe = jnp.pad(obj_cls_W[:DIM], ((0, 0), (0, npad)))
    owr = jnp.pad(obj_cls_W[DIM:], ((0, 0), (0, npad)))
    ob = jnp.pad(obj_cls_b, (0, npad)).reshape(1, N_PAD)
    sp, op_ = _tc_cls(ho, orr, hs, sr, swe, swr, sb, owe, owr, ob)
    return (sp[:, :NUME], op_[:, :NUME])


# speedup vs baseline: 68.2404x; 1.1500x over previous
import functools

import jax
import jax.numpy as jnp
from jax import lax
from jax.experimental import pallas as pl
from jax.experimental.pallas import tpu as pltpu
from jax.experimental.pallas import tpu_sc as plsc

NUME = 10000
DIM = 128
NUMR = 8
R = 2 * NUMR
B = 1024
E = 320000

NC = 2
NS = 16
LANES = 16
NW = NC * NS
CH = 64
GRP = 8
NCH = 160
NGRP = NCH // GRP
CHP = 128
NCHP = 80
E_PAD = NW * CH * NCH
P = 10240
T = R * P
TROWS = T // 128
DEG_STRIPE = T // NS
ACC_STRIPE = P // NS
M_T = 640
NT = P // M_T
N_PAD = 10240
N_T = 512
BW = B // NW


@functools.cache
def _mesh():
    return plsc.VectorSubcoreMesh(core_axis_name="c", subcore_axis_name="s")


def _wid():
    return lax.axis_index("s") * NC + lax.axis_index("c")


def _sc_prep_body(src_h, dst_h, et_h, zdeg_h, idxs_out, idxd_out, deg_out,
                  src_g, dst_g, et_g, ixs_g, ixd_g, ones_v,
                  ico0, ico1, ici0, ici1, dego_sh, degi_sh,
                  so0, so1, si0, si1):
    cid = lax.axis_index("c")
    sid = lax.axis_index("s")
    wid = _wid()
    ico = [ico0, ico1]
    ici = [ici0, ici1]
    so = [so0, so1]
    si = [si0, si1]
    for k in range(CHP // LANES):
        ones_v[pl.ds(k * LANES, LANES)] = jnp.ones((LANES,), jnp.float32)
    st = pl.ds(sid * DEG_STRIPE, DEG_STRIPE)
    pltpu.sync_copy(zdeg_h.at[st], dego_sh.at[st])
    pltpu.sync_copy(zdeg_h.at[st], degi_sh.at[st])
    plsc.subcore_barrier()
    GC = GRP * CHP
    base0 = wid * NCHP * CHP

    def group(gi, carry):
        gsl = pl.ds(base0 + gi * GC, GC)
        pltpu.sync_copy(src_h.at[gsl], src_g)
        pltpu.sync_copy(dst_h.at[gsl], dst_g)
        pltpu.sync_copy(et_h.at[gsl], et_g)

        @plsc.parallel_loop(0, GC, step=LANES, unroll=4)
        def compute(i):
            s = pl.ds(i, LANES)
            ev = et_g[s] * P
            ixs_g[s] = ev + src_g[s]
            ixd_g[s] = ev + dst_g[s]

        pltpu.sync_copy(ixs_g, idxs_out.at[gsl])
        pltpu.sync_copy(ixd_g, idxd_out.at[gsl])
        for k in range(GRP):
            b = k % 2
            if k >= 2:
                pltpu.make_async_copy(
                    ones_v, dego_sh.at[ico[b]], so[b]).wait()
                pltpu.make_async_copy(
                    ones_v, degi_sh.at[ici[b]], si[b]).wait()
            for q in range(CHP // LANES):
                s = pl.ds(q * LANES, LANES)
                ico[b][s] = ixs_g[pl.ds(k * CHP + q * LANES, LANES)]
                ici[b][s] = ixd_g[pl.ds(k * CHP + q * LANES, LANES)]
            pltpu.async_copy(ones_v, dego_sh.at[ico[b]], so[b], add=True)
            pltpu.async_copy(ones_v, degi_sh.at[ici[b]], si[b], add=True)
        for b in range(2):
            pltpu.make_async_copy(ones_v, dego_sh.at[ico[b]], so[b]).wait()
            pltpu.make_async_copy(ones_v, degi_sh.at[ici[b]], si[b]).wait()
        return carry

    lax.fori_loop(0, NCHP // GRP, group, 0)
    plsc.subcore_barrier()
    off = cid * 2 * T + sid * DEG_STRIPE
    pltpu.sync_copy(dego_sh.at[st], deg_out.at[pl.ds(off, DEG_STRIPE)])
    pltpu.sync_copy(degi_sh.at[st], deg_out.at[pl.ds(off + T, DEG_STRIPE)])


@functools.cache
def _sc_prep():
    return pl.kernel(
        _sc_prep_body,
        out_type=(
            jax.ShapeDtypeStruct((E_PAD,), jnp.int32),
            jax.ShapeDtypeStruct((E_PAD,), jnp.int32),
            jax.ShapeDtypeStruct((NC * 2 * T,), jnp.float32),
        ),
        mesh=_mesh(),
        compiler_params=pltpu.CompilerParams(needs_layout_passes=False),
        scratch_types=[
            pltpu.VMEM((GRP * CHP,), jnp.int32),
            pltpu.VMEM((GRP * CHP,), jnp.int32),
            pltpu.VMEM((GRP * CHP,), jnp.int32),
            pltpu.VMEM((GRP * CHP,), jnp.int32),
            pltpu.VMEM((GRP * CHP,), jnp.int32),
            pltpu.VMEM((CHP,), jnp.float32),
            pltpu.VMEM((CHP,), jnp.int32),
            pltpu.VMEM((CHP,), jnp.int32),
            pltpu.VMEM((CHP,), jnp.int32),
            pltpu.VMEM((CHP,), jnp.int32),
            pltpu.VMEM_SHARED((T,), jnp.float32),
            pltpu.VMEM_SHARED((T,), jnp.float32),
            pltpu.SemaphoreType.DMA,
            pltpu.SemaphoreType.DMA,
            pltpu.SemaphoreType.DMA,
            pltpu.SemaphoreType.DMA,
        ],
    )


def _tc_norm_body(deg_ref, no_ref, ni_ref):
    d = deg_ref[...]
    do = d[0, 0] + d[1, 0]
    di = d[0, 1] + d[1, 1]
    rowi = lax.broadcasted_iota(jnp.int32, (TROWS, 128), 0)
    coli = lax.broadcasted_iota(jnp.int32, (TROWS, 128), 1)
    mask = lax.rem(rowi * 128 + coli, P) < NUME
    no_ref[...] = jnp.where(mask, lax.rsqrt(jnp.maximum(do, 1.0)), 0.0)
    ni_ref[...] = jnp.where(mask, lax.rsqrt(jnp.maximum(di, 1.0)), 0.0)


def _tc_norm(degs4):
    return pl.pallas_call(
        _tc_norm_body,
        out_shape=(
            jax.ShapeDtypeStruct((TROWS, 128), jnp.float32),
            jax.ShapeDtypeStruct((TROWS, 128), jnp.float32),
        ),
    )(degs4)


def _sc_coef_body(normo_h, normi_h, idxs_h, idxd_h, c_out,
                  ixs, ixd, cg, a0, a1, b0, b1, sa0, sa1, sb0, sb1):
    wid = _wid()
    av = [a0, a1]
    bv = [b0, b1]
    sa = [sa0, sa1]
    sb = [sb0, sb1]
    GC = GRP * CHP
    base0 = wid * NCHP * CHP

    def group(gi, carry):
        gsl = pl.ds(base0 + gi * GC, GC)
        pltpu.sync_copy(idxs_h.at[gsl], ixs)
        pltpu.sync_copy(idxd_h.at[gsl], ixd)
        for p in range(2):
            pltpu.async_copy(
                normo_h.at[ixs.at[pl.ds(p * CHP, CHP)]], av[p], sa[p])
            pltpu.async_copy(
                normi_h.at[ixd.at[pl.ds(p * CHP, CHP)]], bv[p], sb[p])
        for k in range(GRP):
            b = k % 2
            ksl = pl.ds(k * CHP, CHP)
            pltpu.make_async_copy(
                normo_h.at[ixs.at[ksl]], av[b], sa[b]).wait()
            pltpu.make_async_copy(
                normi_h.at[ixd.at[ksl]], bv[b], sb[b]).wait()
            for q in range(CHP // LANES):
                s = pl.ds(q * LANES, LANES)
                cg[pl.ds(k * CHP + q * LANES, LANES)] = av[b][s] * bv[b][s]
            if k + 2 < GRP:
                nsl = pl.ds((k + 2) * CHP, CHP)
                pltpu.async_copy(normo_h.at[ixs.at[nsl]], av[b], sa[b])
                pltpu.async_copy(normi_h.at[ixd.at[nsl]], bv[b], sb[b])
        pltpu.sync_copy(cg, c_out.at[gsl])
        return carry

    lax.fori_loop(0, NCHP // GRP, group, 0)


@functools.cache
def _sc_coef():
    return pl.kernel(
        _sc_coef_body,
        out_type=jax.ShapeDtypeStruct((E_PAD,), jnp.float32),
        mesh=_mesh(),
        compiler_params=pltpu.CompilerParams(needs_layout_passes=False),
        scratch_types=[
            pltpu.VMEM((GRP * CHP,), jnp.int32),
            pltpu.VMEM((GRP * CHP,), jnp.int32),
            pltpu.VMEM((GRP * CHP,), jnp.float32),
            pltpu.VMEM((CHP,), jnp.float32),
            pltpu.VMEM((CHP,), jnp.float32),
            pltpu.VMEM((CHP,), jnp.float32),
            pltpu.VMEM((CHP,), jnp.float32),
            pltpu.SemaphoreType.DMA,
            pltpu.SemaphoreType.DMA,
            pltpu.SemaphoreType.DMA,
            pltpu.SemaphoreType.DMA,
        ],
    )


def _tc_g0_body(h_ref, w_ref, g_ref):
    g_ref[...] = jnp.dot(h_ref[...], w_ref[0],
                         preferred_element_type=jnp.float32)


def _tc_g0(h, w):
    return pl.pallas_call(
        _tc_g0_body,
        grid=(R, NT),
        in_specs=[
            pl.BlockSpec((M_T, DIM), lambda r, i: (i, 0)),
            pl.BlockSpec((1, DIM, DIM), lambda r, i: (r, 0, 0)),
        ],
        out_specs=pl.BlockSpec((M_T, DIM), lambda r, i: (r * NT + i, 0)),
        out_shape=jax.ShapeDtypeStruct((R * P, DIM), jnp.float32),
    )(h, w)


def _tc_g1_body(p0_ref, p1_ref, cb_ref, w_ref, g_ref):
    bm = jnp.sum(cb_ref[...], axis=0, keepdims=True) * (1.0 / R)
    h = (p0_ref[0] + p1_ref[0]) * (1.0 / R) + bm
    g_ref[...] = jnp.dot(h, w_ref[0], preferred_element_type=jnp.float32)


def _tc_g1(parts, cb, w):
    return pl.pallas_call(
        _tc_g1_body,
        grid=(R, NT),
        in_specs=[
            pl.BlockSpec((1, M_T, DIM), lambda r, i: (0, i, 0)),
            pl.BlockSpec((1, M_T, DIM), lambda r, i: (1, i, 0)),
            pl.BlockSpec((R, DIM), lambda r, i: (0, 0)),
            pl.BlockSpec((1, DIM, DIM), lambda r, i: (r, 0, 0)),
        ],
        out_specs=pl.BlockSpec((M_T, DIM), lambda r, i: (r * NT + i, 0)),
        out_shape=jax.ShapeDtypeStruct((R * P, DIM), jnp.float32),
    )(parts, parts, cb, w)


def _sc_scatter_body(g_h, idx_h, c_h, dst_h, zrow_h, part_out,
                     idx8, c8, dst8, dc0, dc1, rg0, rg1, rs0, rs1, accum_sh,
                     gs0, gs1, ss0, ss1):
    cid = lax.axis_index("c")
    sid = lax.axis_index("s")
    wid = _wid()
    st = pl.ds(sid * ACC_STRIPE, ACC_STRIPE)
    pltpu.sync_copy(zrow_h.at[st], accum_sh.at[st])
    plsc.subcore_barrier()
    dc = [dc0, dc1]
    rg = [rg0, rg1]
    rs = [rs0, rs1]
    gs = [gs0, gs1]
    ss = [ss0, ss1]
    col0 = lax.iota(jnp.int32, LANES)
    base0 = wid * NCH * CH

    def group(gi, carry):
        gbase = base0 + gi * GRP * CH
        gsl = pl.ds(gbase, GRP * CH)
        pltpu.sync_copy(idx_h.at[gsl], idx8)
        pltpu.sync_copy(c_h.at[gsl], c8)
        pltpu.sync_copy(dst_h.at[gsl], dst8)
        pltpu.async_copy(g_h.at[idx8.at[pl.ds(0, CH)]], rg[0], gs[0])
        pltpu.async_copy(g_h.at[idx8.at[pl.ds(CH, CH)]], rg[1], gs[1])
        for k in range(GRP):
            b = k % 2
            pltpu.make_async_copy(
                g_h.at[idx8.at[pl.ds(k * CH, CH)]], rg[b], gs[b]).wait()
            if k >= 2:
                pltpu.make_async_copy(rs[b], accum_sh.at[dc[b]], ss[b]).wait()
            for q in range(CH // LANES):
                dc[b][pl.ds(q * LANES, LANES)] = dst8[
                    pl.ds(k * CH + q * LANES, LANES)]
            kk = jnp.full((LANES,), k * CH, jnp.int32)

            @plsc.parallel_loop(0, CH, unroll=4)
            def scale(i, _b=b, _kk=kk):
                ii = jnp.zeros((LANES,), jnp.int32) + i
                cb = plsc.load_gather(c8, [_kk + ii])
                for q in range(DIM // LANES):
                    idxs = [ii, col0 + q * LANES]
                    v = plsc.load_gather(rg[_b], idxs)
                    plsc.store_scatter(rs[_b], idxs, v * cb)
            if k + 2 < GRP:
                pltpu.async_copy(
                    g_h.at[idx8.at[pl.ds((k + 2) * CH, CH)]], rg[b], gs[b])
            pltpu.async_copy(rs[b], accum_sh.at[dc[b]], ss[b], add=True)
        pltpu.make_async_copy(rs[0], accum_sh.at[dc[0]], ss[0]).wait()
        pltpu.make_async_copy(rs[1], accum_sh.at[dc[1]], ss[1]).wait()
        return carry

    lax.fori_loop(0, NGRP, group, 0)
    plsc.subcore_barrier()
    pltpu.sync_copy(accum_sh.at[st], part_out.at[cid, st])


@functools.cache
def _sc_scatter():
    return pl.kernel(
        _sc_scatter_body,
        out_type=jax.ShapeDtypeStruct((NC, P, DIM), jnp.float32),
        mesh=_mesh(),
        compiler_params=pltpu.CompilerParams(needs_layout_passes=False),
        scratch_types=[
            pltpu.VMEM((GRP * CH,), jnp.int32),
            pltpu.VMEM((GRP * CH,), jnp.float32),
            pltpu.VMEM((GRP * CH,), jnp.int32),
            pltpu.VMEM((CH,), jnp.int32),
            pltpu.VMEM((CH,), jnp.int32),
            pltpu.VMEM((CH, DIM), jnp.float32),
            pltpu.VMEM((CH, DIM), jnp.float32),
            pltpu.VMEM((CH, DIM), jnp.float32),
            pltpu.VMEM((CH, DIM), jnp.float32),
            pltpu.VMEM_SHARED((P, DIM), jnp.float32),
            pltpu.SemaphoreType.DMA,
            pltpu.SemaphoreType.DMA,
            pltpu.SemaphoreType.DMA,
            pltpu.SemaphoreType.DMA,
        ],
    )


def _tc_h_body(p0_ref, p1_ref, cb_ref, h_ref):
    bm = jnp.sum(cb_ref[...], axis=0, keepdims=True) * (1.0 / R)
    h_ref[...] = (p0_ref[0] + p1_ref[0]) * (1.0 / R) + bm


def _tc_h(parts, cb):
    return pl.pallas_call(
        _tc_h_body,
        grid=(NT,),
        in_specs=[
            pl.BlockSpec((1, M_T, DIM), lambda i: (0, i, 0)),
            pl.BlockSpec((1, M_T, DIM), lambda i: (1, i, 0)),
            pl.BlockSpec((R, DIM), lambda i: (0, 0)),
        ],
        out_specs=pl.BlockSpec((M_T, DIM), lambda i: (i, 0)),
        out_shape=jax.ShapeDtypeStruct((P, DIM), jnp.float32),
    )(parts, parts, cb)


def _sc_gather_body(h_h, sub_h, obj_h, rel_h, sre_h, ore_h,
                    hs_out, ho_out, sr_out, or_out,
                    i_v, hrows_v, rrows_v, sem):
    wid = _wid()
    sl = pl.ds(wid * BW, BW)
    pltpu.sync_copy(sub_h.at[sl], i_v)
    pltpu.async_copy(h_h.at[i_v], hrows_v, sem).wait()
    pltpu.sync_copy(hrows_v, hs_out.at[sl])
    pltpu.sync_copy(obj_h.at[sl], i_v)
    pltpu.async_copy(h_h.at[i_v], hrows_v, sem).wait()
    pltpu.sync_copy(hrows_v, ho_out.at[sl])
    pltpu.sync_copy(rel_h.at[sl], i_v)
    pltpu.async_copy(sre_h.at[i_v], rrows_v, sem).wait()
    pltpu.sync_copy(rrows_v, sr_out.at[sl])
    pltpu.async_copy(ore_h.at[i_v], rrows_v, sem).wait()
    pltpu.sync_copy(rrows_v, or_out.at[sl])


@functools.cache
def _sc_gather():
    return pl.kernel(
        _sc_gather_body,
        out_type=(
            jax.ShapeDtypeStruct((B, DIM), jnp.float32),
            jax.ShapeDtypeStruct((B, DIM), jnp.float32),
            jax.ShapeDtypeStruct((B, DIM), jnp.float32),
            jax.ShapeDtypeStruct((B, DIM), jnp.float32),
        ),
        mesh=_mesh(),
        compiler_params=pltpu.CompilerParams(needs_layout_passes=False),
        scratch_types=[
            pltpu.VMEM((BW,), jnp.int32),
            pltpu.VMEM((BW, DIM), jnp.float32),
            pltpu.VMEM((BW, DIM), jnp.float32),
            pltpu.SemaphoreType.DMA,
        ],
    )


def _tc_cls_body(ho_ref, or_ref, hs_ref, sr_ref,
                 swe_ref, swr_ref, sb_ref, owe_ref, owr_ref, ob_ref,
                 sp_ref, op_ref):
    sp_ref[...] = (jnp.dot(ho_ref[...], swe_ref[...],
                           preferred_element_type=jnp.float32)
                   + jnp.dot(or_ref[...], swr_ref[...],
                             preferred_element_type=jnp.float32)
                   + sb_ref[...])
    op_ref[...] = (jnp.dot(hs_ref[...], owe_ref[...],
                           preferred_element_type=jnp.float32)
                   + jnp.dot(sr_ref[...], owr_ref[...],
                             preferred_element_type=jnp.float32)
                   + ob_ref[...])


def _tc_cls(ho, orr, hs, sr, swe, swr, sb, owe, owr, ob):
    return pl.pallas_call(
        _tc_cls_body,
        grid=(N_PAD // N_T,),
        in_specs=[
            pl.BlockSpec((B, DIM), lambda n: (0, 0)),
            pl.BlockSpec((B, 32), lambda n: (0, 0)),
            pl.BlockSpec((B, DIM), lambda n: (0, 0)),
            pl.BlockSpec((B, 32), lambda n: (0, 0)),
            pl.BlockSpec((DIM, N_T), lambda n: (0, n)),
            pl.BlockSpec((32, N_T), lambda n: (0, n)),
            pl.BlockSpec((1, N_T), lambda n: (0, n)),
            pl.BlockSpec((DIM, N_T), lambda n: (0, n)),
            pl.BlockSpec((32, N_T), lambda n: (0, n)),
            pl.BlockSpec((1, N_T), lambda n: (0, n)),
        ],
        out_specs=(
            pl.BlockSpec((B, N_T), lambda n: (0, n)),
            pl.BlockSpec((B, N_T), lambda n: (0, n)),
        ),
        out_shape=(
            jax.ShapeDtypeStruct((B, N_PAD), jnp.float32),
            jax.ShapeDtypeStruct((B, N_PAD), jnp.float32),
        ),
    )(ho, orr, hs, sr, swe, swr, sb, owe, owr, ob)


def kernel(sub, obj, rel, edge_index, etype, ts, entity_emb, sub_rel_emb,
           obj_rel_emb, conv_W, conv_b, obj_cls_W, obj_cls_b, sub_cls_W,
           sub_cls_b):
    src = edge_index[0].astype(jnp.int32)
    dst = edge_index[1].astype(jnp.int32)
    et = etype.astype(jnp.int32)
    pad = E_PAD - E
    padv = NUME + (jnp.arange(pad, dtype=jnp.int32) % (P - NUME))
    src_p = jnp.concatenate([src, padv])
    dst_p = jnp.concatenate([dst, padv])
    et_p = jnp.concatenate([et, jnp.zeros((pad,), jnp.int32)])
    zdeg = jnp.zeros((T,), jnp.float32)
    zrow = jnp.zeros((P, DIM), jnp.float32)

    idx_s, idx_d, degs = _sc_prep()(src_p, dst_p, et_p, zdeg)
    normo, normi = _tc_norm(degs.reshape(NC, 2, TROWS, 128))
    c = _sc_coef()(normo.reshape(T), normi.reshape(T), idx_s, idx_d)

    emb_p = jnp.pad(entity_emb, ((0, P - NUME), (0, 0)))

    g0 = _tc_g0(emb_p, conv_W[0])
    parts0 = _sc_scatter()(g0, idx_s, c, dst_p, zrow)
    g1 = _tc_g1(parts0, conv_b[0], conv_W[1])
    parts1 = _sc_scatter()(g1, idx_s, c, dst_p, zrow)
    h2 = _tc_h(parts1, conv_b[1])

    srp = jnp.pad(sub_rel_emb, ((0, 0), (0, DIM - 32)))
    orp = jnp.pad(obj_rel_emb, ((0, 0), (0, DIM - 32)))
    hs, ho, sr, orr = _sc_gather()(h2, sub.astype(jnp.int32),
                                   obj.astype(jnp.int32),
                                   rel.astype(jnp.int32), srp, orp)
    sr = sr[:, :32]
    orr = orr[:, :32]

    npad = N_PAD - NUME
    swe = jnp.pad(sub_cls_W[:DIM], ((0, 0), (0, npad)))
    swr = jnp.pad(sub_cls_W[DIM:], ((0, 0), (0, npad)))
    sb = jnp.pad(sub_cls_b, (0, npad)).reshape(1, N_PAD)
    owe = jnp.pad(obj_cls_W[:DIM], ((0, 0), (0, npad)))
    owr = jnp.pad(obj_cls_W[DIM:], ((0, 0), (0, npad)))
    ob = jnp.pad(obj_cls_b, (0, npad)).reshape(1, N_PAD)
    sp, op_ = _tc_cls(ho, orr, hs, sr, swe, swr, sb, owe, owr, ob)
    return (sp[:, :NUME], op_[:, :NUME])


# speedup vs baseline: 68.2462x; 1.0001x over previous
import functools

import jax
import jax.numpy as jnp
from jax import lax
from jax.experimental import pallas as pl
from jax.experimental.pallas import tpu as pltpu
from jax.experimental.pallas import tpu_sc as plsc

NUME = 10000
DIM = 128
NUMR = 8
R = 2 * NUMR
B = 1024
E = 320000

NC = 2
NS = 16
LANES = 16
NW = NC * NS
CH = 64
GRP = 8
NCH = 160
NGRP = NCH // GRP
CHP = 128
NCHP = 80
E_PAD = NW * CH * NCH
P = 10240
T = R * P
TROWS = T // 128
DEG_STRIPE = T // NS
ACC_STRIPE = P // NS
M_T = 640
NT = P // M_T
N_PAD = 10240
N_T = 512
BW = B // NW


@functools.cache
def _mesh():
    return plsc.VectorSubcoreMesh(core_axis_name="c", subcore_axis_name="s")


def _wid():
    return lax.axis_index("s") * NC + lax.axis_index("c")


def _sc_prep_body(src_h, dst_h, et_h, zdeg_h, idxs_out, idxd_out, deg_out,
                  src_g, dst_g, et_g, ixs_g, ixd_g, ones_v,
                  ico0, ico1, ici0, ici1, dego_sh, degi_sh,
                  so0, so1, si0, si1):
    cid = lax.axis_index("c")
    sid = lax.axis_index("s")
    wid = _wid()
    ico = [ico0, ico1]
    ici = [ici0, ici1]
    so = [so0, so1]
    si = [si0, si1]
    for k in range(CHP // LANES):
        ones_v[pl.ds(k * LANES, LANES)] = jnp.ones((LANES,), jnp.float32)
    st = pl.ds(sid * DEG_STRIPE, DEG_STRIPE)
    pltpu.sync_copy(zdeg_h.at[st], dego_sh.at[st])
    pltpu.sync_copy(zdeg_h.at[st], degi_sh.at[st])
    plsc.subcore_barrier()
    GC = GRP * CHP
    base0 = wid * NCHP * CHP

    def group(gi, carry):
        gsl = pl.ds(base0 + gi * GC, GC)
        pltpu.sync_copy(src_h.at[gsl], src_g)
        pltpu.sync_copy(dst_h.at[gsl], dst_g)
        pltpu.sync_copy(et_h.at[gsl], et_g)

        @plsc.parallel_loop(0, GC, step=LANES, unroll=4)
        def compute(i):
            s = pl.ds(i, LANES)
            ev = et_g[s] * P
            ixs_g[s] = ev + src_g[s]
            ixd_g[s] = ev + dst_g[s]

        pltpu.sync_copy(ixs_g, idxs_out.at[gsl])
        pltpu.sync_copy(ixd_g, idxd_out.at[gsl])
        for k in range(GRP):
            b = k % 2
            if k >= 2:
                pltpu.make_async_copy(
                    ones_v, dego_sh.at[ico[b]], so[b]).wait()
                pltpu.make_async_copy(
                    ones_v, degi_sh.at[ici[b]], si[b]).wait()
            for q in range(CHP // LANES):
                s = pl.ds(q * LANES, LANES)
                ico[b][s] = ixs_g[pl.ds(k * CHP + q * LANES, LANES)]
                ici[b][s] = ixd_g[pl.ds(k * CHP + q * LANES, LANES)]
            pltpu.async_copy(ones_v, dego_sh.at[ico[b]], so[b], add=True)
            pltpu.async_copy(ones_v, degi_sh.at[ici[b]], si[b], add=True)
        for b in range(2):
            pltpu.make_async_copy(ones_v, dego_sh.at[ico[b]], so[b]).wait()
            pltpu.make_async_copy(ones_v, degi_sh.at[ici[b]], si[b]).wait()
        return carry

    lax.fori_loop(0, NCHP // GRP, group, 0)
    plsc.subcore_barrier()
    off = cid * 2 * T + sid * DEG_STRIPE
    pltpu.sync_copy(dego_sh.at[st], deg_out.at[pl.ds(off, DEG_STRIPE)])
    pltpu.sync_copy(degi_sh.at[st], deg_out.at[pl.ds(off + T, DEG_STRIPE)])


@functools.cache
def _sc_prep():
    return pl.kernel(
        _sc_prep_body,
        out_type=(
            jax.ShapeDtypeStruct((E_PAD,), jnp.int32),
            jax.ShapeDtypeStruct((E_PAD,), jnp.int32),
            jax.ShapeDtypeStruct((NC * 2 * T,), jnp.float32),
        ),
        mesh=_mesh(),
        compiler_params=pltpu.CompilerParams(needs_layout_passes=False, use_tc_tiling_on_sc=True),
        scratch_types=[
            pltpu.VMEM((GRP * CHP,), jnp.int32),
            pltpu.VMEM((GRP * CHP,), jnp.int32),
            pltpu.VMEM((GRP * CHP,), jnp.int32),
            pltpu.VMEM((GRP * CHP,), jnp.int32),
            pltpu.VMEM((GRP * CHP,), jnp.int32),
            pltpu.VMEM((CHP,), jnp.float32),
            pltpu.VMEM((CHP,), jnp.int32),
            pltpu.VMEM((CHP,), jnp.int32),
            pltpu.VMEM((CHP,), jnp.int32),
            pltpu.VMEM((CHP,), jnp.int32),
            pltpu.VMEM_SHARED((T,), jnp.float32),
            pltpu.VMEM_SHARED((T,), jnp.float32),
            pltpu.SemaphoreType.DMA,
            pltpu.SemaphoreType.DMA,
            pltpu.SemaphoreType.DMA,
            pltpu.SemaphoreType.DMA,
        ],
    )


def _tc_norm_body(deg_ref, no_ref, ni_ref):
    d = deg_ref[...]
    do = d[0, 0] + d[1, 0]
    di = d[0, 1] + d[1, 1]
    rowi = lax.broadcasted_iota(jnp.int32, (TROWS, 128), 0)
    coli = lax.broadcasted_iota(jnp.int32, (TROWS, 128), 1)
    mask = lax.rem(rowi * 128 + coli, P) < NUME
    no_ref[...] = jnp.where(mask, lax.rsqrt(jnp.maximum(do, 1.0)), 0.0)
    ni_ref[...] = jnp.where(mask, lax.rsqrt(jnp.maximum(di, 1.0)), 0.0)


def _tc_norm(degs4):
    return pl.pallas_call(
        _tc_norm_body,
        out_shape=(
            jax.ShapeDtypeStruct((TROWS, 128), jnp.float32),
            jax.ShapeDtypeStruct((TROWS, 128), jnp.float32),
        ),
    )(degs4)


def _sc_coef_body(normo_h, normi_h, idxs_h, idxd_h, c_out,
                  ixs, ixd, cg, a0, a1, b0, b1, sa0, sa1, sb0, sb1):
    wid = _wid()
    av = [a0, a1]
    bv = [b0, b1]
    sa = [sa0, sa1]
    sb = [sb0, sb1]
    GC = GRP * CHP
    base0 = wid * NCHP * CHP

    def group(gi, carry):
        gsl = pl.ds(base0 + gi * GC, GC)
        pltpu.sync_copy(idxs_h.at[gsl], ixs)
        pltpu.sync_copy(idxd_h.at[gsl], ixd)
        for p in range(2):
            pltpu.async_copy(
                normo_h.at[ixs.at[pl.ds(p * CHP, CHP)]], av[p], sa[p])
            pltpu.async_copy(
                normi_h.at[ixd.at[pl.ds(p * CHP, CHP)]], bv[p], sb[p])
        for k in range(GRP):
            b = k % 2
            ksl = pl.ds(k * CHP, CHP)
            pltpu.make_async_copy(
                normo_h.at[ixs.at[ksl]], av[b], sa[b]).wait()
            pltpu.make_async_copy(
                normi_h.at[ixd.at[ksl]], bv[b], sb[b]).wait()
            for q in range(CHP // LANES):
                s = pl.ds(q * LANES, LANES)
                cg[pl.ds(k * CHP + q * LANES, LANES)] = av[b][s] * bv[b][s]
            if k + 2 < GRP:
                nsl = pl.ds((k + 2) * CHP, CHP)
                pltpu.async_copy(normo_h.at[ixs.at[nsl]], av[b], sa[b])
                pltpu.async_copy(normi_h.at[ixd.at[nsl]], bv[b], sb[b])
        pltpu.sync_copy(cg, c_out.at[gsl])
        return carry

    lax.fori_loop(0, NCHP // GRP, group, 0)


@functools.cache
def _sc_coef():
    return pl.kernel(
        _sc_coef_body,
        out_type=jax.ShapeDtypeStruct((E_PAD,), jnp.float32),
        mesh=_mesh(),
        compiler_params=pltpu.CompilerParams(needs_layout_passes=False, use_tc_tiling_on_sc=True),
        scratch_types=[
            pltpu.VMEM((GRP * CHP,), jnp.int32),
            pltpu.VMEM((GRP * CHP,), jnp.int32),
            pltpu.VMEM((GRP * CHP,), jnp.float32),
            pltpu.VMEM((CHP,), jnp.float32),
            pltpu.VMEM((CHP,), jnp.float32),
            pltpu.VMEM((CHP,), jnp.float32),
            pltpu.VMEM((CHP,), jnp.float32),
            pltpu.SemaphoreType.DMA,
            pltpu.SemaphoreType.DMA,
            pltpu.SemaphoreType.DMA,
            pltpu.SemaphoreType.DMA,
        ],
    )


def _tc_g0_body(h_ref, w_ref, g_ref):
    g_ref[...] = jnp.dot(h_ref[...], w_ref[0],
                         preferred_element_type=jnp.float32)


def _tc_g0(h, w):
    return pl.pallas_call(
        _tc_g0_body,
        grid=(R, NT),
        in_specs=[
            pl.BlockSpec((M_T, DIM), lambda r, i: (i, 0)),
            pl.BlockSpec((1, DIM, DIM), lambda r, i: (r, 0, 0)),
        ],
        out_specs=pl.BlockSpec((M_T, DIM), lambda r, i: (r * NT + i, 0)),
        out_shape=jax.ShapeDtypeStruct((R * P, DIM), jnp.float32),
    )(h, w)


def _tc_g1_body(p0_ref, p1_ref, cb_ref, w_ref, g_ref):
    bm = jnp.sum(cb_ref[...], axis=0, keepdims=True) * (1.0 / R)
    h = (p0_ref[0] + p1_ref[0]) * (1.0 / R) + bm
    g_ref[...] = jnp.dot(h, w_ref[0], preferred_element_type=jnp.float32)


def _tc_g1(parts, cb, w):
    return pl.pallas_call(
        _tc_g1_body,
        grid=(R, NT),
        in_specs=[
            pl.BlockSpec((1, M_T, DIM), lambda r, i: (0, i, 0)),
            pl.BlockSpec((1, M_T, DIM), lambda r, i: (1, i, 0)),
            pl.BlockSpec((R, DIM), lambda r, i: (0, 0)),
            pl.BlockSpec((1, DIM, DIM), lambda r, i: (r, 0, 0)),
        ],
        out_specs=pl.BlockSpec((M_T, DIM), lambda r, i: (r * NT + i, 0)),
        out_shape=jax.ShapeDtypeStruct((R * P, DIM), jnp.float32),
    )(parts, parts, cb, w)


def _sc_scatter_body(g_h, idx_h, c_h, dst_h, zrow_h, part_out,
                     idx8, c8, dst8, dc0, dc1, rg0, rg1, rs0, rs1, accum_sh,
                     gs0, gs1, ss0, ss1):
    cid = lax.axis_index("c")
    sid = lax.axis_index("s")
    wid = _wid()
    st = pl.ds(sid * ACC_STRIPE, ACC_STRIPE)
    pltpu.sync_copy(zrow_h.at[st], accum_sh.at[st])
    plsc.subcore_barrier()
    dc = [dc0, dc1]
    rg = [rg0, rg1]
    rs = [rs0, rs1]
    gs = [gs0, gs1]
    ss = [ss0, ss1]
    col0 = lax.iota(jnp.int32, LANES)
    base0 = wid * NCH * CH

    def group(gi, carry):
        gbase = base0 + gi * GRP * CH
        gsl = pl.ds(gbase, GRP * CH)
        pltpu.sync_copy(idx_h.at[gsl], idx8)
        pltpu.sync_copy(c_h.at[gsl], c8)
        pltpu.sync_copy(dst_h.at[gsl], dst8)
        pltpu.async_copy(g_h.at[idx8.at[pl.ds(0, CH)]], rg[0], gs[0])
        pltpu.async_copy(g_h.at[idx8.at[pl.ds(CH, CH)]], rg[1], gs[1])
        for k in range(GRP):
            b = k % 2
            pltpu.make_async_copy(
                g_h.at[idx8.at[pl.ds(k * CH, CH)]], rg[b], gs[b]).wait()
            if k >= 2:
                pltpu.make_async_copy(rs[b], accum_sh.at[dc[b]], ss[b]).wait()
            for q in range(CH // LANES):
                dc[b][pl.ds(q * LANES, LANES)] = dst8[
                    pl.ds(k * CH + q * LANES, LANES)]
            kk = jnp.full((LANES,), k * CH, jnp.int32)

            @plsc.parallel_loop(0, CH, unroll=4)
            def scale(i, _b=b, _kk=kk):
                ii = jnp.zeros((LANES,), jnp.int32) + i
                cb = plsc.load_gather(c8, [_kk + ii])
                for q in range(DIM // LANES):
                    idxs = [ii, col0 + q * LANES]
                    v = plsc.load_gather(rg[_b], idxs)
                    plsc.store_scatter(rs[_b], idxs, v * cb)
            if k + 2 < GRP:
                pltpu.async_copy(
                    g_h.at[idx8.at[pl.ds((k + 2) * CH, CH)]], rg[b], gs[b])
            pltpu.async_copy(rs[b], accum_sh.at[dc[b]], ss[b], add=True)
        pltpu.make_async_copy(rs[0], accum_sh.at[dc[0]], ss[0]).wait()
        pltpu.make_async_copy(rs[1], accum_sh.at[dc[1]], ss[1]).wait()
        return carry

    lax.fori_loop(0, NGRP, group, 0)
    plsc.subcore_barrier()
    pltpu.sync_copy(accum_sh.at[st], part_out.at[cid, st])


@functools.cache
def _sc_scatter():
    return pl.kernel(
        _sc_scatter_body,
        out_type=jax.ShapeDtypeStruct((NC, P, DIM), jnp.float32),
        mesh=_mesh(),
        compiler_params=pltpu.CompilerParams(needs_layout_passes=False, use_tc_tiling_on_sc=True),
        scratch_types=[
            pltpu.VMEM((GRP * CH,), jnp.int32),
            pltpu.VMEM((GRP * CH,), jnp.float32),
            pltpu.VMEM((GRP * CH,), jnp.int32),
            pltpu.VMEM((CH,), jnp.int32),
            pltpu.VMEM((CH,), jnp.int32),
            pltpu.VMEM((CH, DIM), jnp.float32),
            pltpu.VMEM((CH, DIM), jnp.float32),
            pltpu.VMEM((CH, DIM), jnp.float32),
            pltpu.VMEM((CH, DIM), jnp.float32),
            pltpu.VMEM_SHARED((P, DIM), jnp.float32),
            pltpu.SemaphoreType.DMA,
            pltpu.SemaphoreType.DMA,
            pltpu.SemaphoreType.DMA,
            pltpu.SemaphoreType.DMA,
        ],
    )


def _tc_h_body(p0_ref, p1_ref, cb_ref, h_ref):
    bm = jnp.sum(cb_ref[...], axis=0, keepdims=True) * (1.0 / R)
    h_ref[...] = (p0_ref[0] + p1_ref[0]) * (1.0 / R) + bm


def _tc_h(parts, cb):
    return pl.pallas_call(
        _tc_h_body,
        grid=(NT,),
        in_specs=[
            pl.BlockSpec((1, M_T, DIM), lambda i: (0, i, 0)),
            pl.BlockSpec((1, M_T, DIM), lambda i: (1, i, 0)),
            pl.BlockSpec((R, DIM), lambda i: (0, 0)),
        ],
        out_specs=pl.BlockSpec((M_T, DIM), lambda i: (i, 0)),
        out_shape=jax.ShapeDtypeStruct((P, DIM), jnp.float32),
    )(parts, parts, cb)


def _sc_gather_body(h_h, sub_h, obj_h, rel_h, sre_h, ore_h,
                    hs_out, ho_out, sr_out, or_out,
                    i_v, hrows_v, rrows_v, sem):
    wid = _wid()
    sl = pl.ds(wid * BW, BW)
    pltpu.sync_copy(sub_h.at[sl], i_v)
    pltpu.async_copy(h_h.at[i_v], hrows_v, sem).wait()
    pltpu.sync_copy(hrows_v, hs_out.at[sl])
    pltpu.sync_copy(obj_h.at[sl], i_v)
    pltpu.async_copy(h_h.at[i_v], hrows_v, sem).wait()
    pltpu.sync_copy(hrows_v, ho_out.at[sl])
    pltpu.sync_copy(rel_h.at[sl], i_v)
    pltpu.async_copy(sre_h.at[i_v], rrows_v, sem).wait()
    pltpu.sync_copy(rrows_v, sr_out.at[sl])
    pltpu.async_copy(ore_h.at[i_v], rrows_v, sem).wait()
    pltpu.sync_copy(rrows_v, or_out.at[sl])


@functools.cache
def _sc_gather():
    return pl.kernel(
        _sc_gather_body,
        out_type=(
            jax.ShapeDtypeStruct((B, DIM), jnp.float32),
            jax.ShapeDtypeStruct((B, DIM), jnp.float32),
            jax.ShapeDtypeStruct((B, DIM), jnp.float32),
            jax.ShapeDtypeStruct((B, DIM), jnp.float32),
        ),
        mesh=_mesh(),
        compiler_params=pltpu.CompilerParams(needs_layout_passes=False, use_tc_tiling_on_sc=True),
        scratch_types=[
            pltpu.VMEM((BW,), jnp.int32),
            pltpu.VMEM((BW, DIM), jnp.float32),
            pltpu.VMEM((BW, DIM), jnp.float32),
            pltpu.SemaphoreType.DMA,
        ],
    )


def _tc_cls_body(ho_ref, or_ref, hs_ref, sr_ref,
                 swe_ref, swr_ref, sb_ref, owe_ref, owr_ref, ob_ref,
                 sp_ref, op_ref):
    sp_ref[...] = (jnp.dot(ho_ref[...], swe_ref[...],
                           preferred_element_type=jnp.float32)
                   + jnp.dot(or_ref[...], swr_ref[...],
                             preferred_element_type=jnp.float32)
                   + sb_ref[...])
    op_ref[...] = (jnp.dot(hs_ref[...], owe_ref[...],
                           preferred_element_type=jnp.float32)
                   + jnp.dot(sr_ref[...], owr_ref[...],
                             preferred_element_type=jnp.float32)
                   + ob_ref[...])


def _tc_cls(ho, orr, hs, sr, swe, swr, sb, owe, owr, ob):
    return pl.pallas_call(
        _tc_cls_body,
        grid=(N_PAD // N_T,),
        in_specs=[
            pl.BlockSpec((B, DIM), lambda n: (0, 0)),
            pl.BlockSpec((B, 32), lambda n: (0, 0)),
            pl.BlockSpec((B, DIM), lambda n: (0, 0)),
            pl.BlockSpec((B, 32), lambda n: (0, 0)),
            pl.BlockSpec((DIM, N_T), lambda n: (0, n)),
            pl.BlockSpec((32, N_T), lambda n: (0, n)),
            pl.BlockSpec((1, N_T), lambda n: (0, n)),
            pl.BlockSpec((DIM, N_T), lambda n: (0, n)),
            pl.BlockSpec((32, N_T), lambda n: (0, n)),
            pl.BlockSpec((1, N_T), lambda n: (0, n)),
        ],
        out_specs=(
            pl.BlockSpec((B, N_T), lambda n: (0, n)),
            pl.BlockSpec((B, N_T), lambda n: (0, n)),
        ),
        out_shape=(
            jax.ShapeDtypeStruct((B, N_PAD), jnp.float32),
            jax.ShapeDtypeStruct((B, N_PAD), jnp.float32),
        ),
    )(ho, orr, hs, sr, swe, swr, sb, owe, owr, ob)


def kernel(sub, obj, rel, edge_index, etype, ts, entity_emb, sub_rel_emb,
           obj_rel_emb, conv_W, conv_b, obj_cls_W, obj_cls_b, sub_cls_W,
           sub_cls_b):
    src = edge_index[0].astype(jnp.int32)
    dst = edge_index[1].astype(jnp.int32)
    et = etype.astype(jnp.int32)
    pad = E_PAD - E
    padv = NUME + (jnp.arange(pad, dtype=jnp.int32) % (P - NUME))
    src_p = jnp.concatenate([src, padv])
    dst_p = jnp.concatenate([dst, padv])
    et_p = jnp.concatenate([et, jnp.zeros((pad,), jnp.int32)])
    zdeg = jnp.zeros((T,), jnp.float32)
    zrow = jnp.zeros((P, DIM), jnp.float32)

    idx_s, idx_d, degs = _sc_prep()(src_p, dst_p, et_p, zdeg)
    normo, normi = _tc_norm(degs.reshape(NC, 2, TROWS, 128))
    c = _sc_coef()(normo.reshape(T), normi.reshape(T), idx_s, idx_d)

    emb_p = jnp.pad(entity_emb, ((0, P - NUME), (0, 0)))

    g0 = _tc_g0(emb_p, conv_W[0])
    parts0 = _sc_scatter()(g0, idx_s, c, dst_p, zrow)
    g1 = _tc_g1(parts0, conv_b[0], conv_W[1])
    parts1 = _sc_scatter()(g1, idx_s, c, dst_p, zrow)
    h2 = _tc_h(parts1, conv_b[1])

    srp = jnp.pad(sub_rel_emb, ((0, 0), (0, DIM - 32)))
    orp = jnp.pad(obj_rel_emb, ((0, 0), (0, DIM - 32)))
    hs, ho, sr, orr = _sc_gather()(h2, sub.astype(jnp.int32),
                                   obj.astype(jnp.int32),
                                   rel.astype(jnp.int32), srp, orp)
    sr = sr[:, :32]
    orr = orr[:, :32]

    npad = N_PAD - NUME
    swe = jnp.pad(sub_cls_W[:DIM], ((0, 0), (0, npad)))
    swr = jnp.pad(sub_cls_W[DIM:], ((0, 0), (0, npad)))
    sb = jnp.pad(sub_cls_b, (0, npad)).reshape(1, N_PAD)
    owe = jnp.pad(obj_cls_W[:DIM], ((0, 0), (0, npad)))
    owr = jnp.pad(obj_cls_W[DIM:], ((0, 0), (0, npad)))
    ob = jnp.pad(obj_cls_b, (0, npad)).reshape(1, N_PAD)
    sp, op_ = _tc_cls(ho, orr, hs, sr, swe, swr, sb, owe, owr, ob)
    return (sp[:, :NUME], op_[:, :NUME])


# speedup vs baseline: 70.0036x; 1.0258x over previous
import functools

import jax
import jax.numpy as jnp
from jax import lax
from jax.experimental import pallas as pl
from jax.experimental.pallas import tpu as pltpu
from jax.experimental.pallas import tpu_sc as plsc

NUME = 10000
DIM = 128
NUMR = 8
R = 2 * NUMR
B = 1024
E = 320000

NC = 2
NS = 16
LANES = 16
NW = NC * NS
CH = 80
GRP = 8
NCH = 128
NGRP = NCH // GRP
CHP = 128
NCHP = 80
E_PAD = NW * CH * NCH
P = 10240
T = R * P
TROWS = T // 128
DEG_STRIPE = T // NS
ACC_STRIPE = P // NS
M_T = 640
NT = P // M_T
N_PAD = 10240
N_T = 512
BW = B // NW


@functools.cache
def _mesh():
    return plsc.VectorSubcoreMesh(core_axis_name="c", subcore_axis_name="s")


def _wid():
    return lax.axis_index("s") * NC + lax.axis_index("c")


def _sc_prep_body(src_h, dst_h, et_h, zdeg_h, idxs_out, idxd_out, deg_out,
                  src_g, dst_g, et_g, ixs_g, ixd_g, ones_v,
                  ico0, ico1, ici0, ici1, dego_sh, degi_sh,
                  so0, so1, si0, si1):
    cid = lax.axis_index("c")
    sid = lax.axis_index("s")
    wid = _wid()
    ico = [ico0, ico1]
    ici = [ici0, ici1]
    so = [so0, so1]
    si = [si0, si1]
    for k in range(CHP // LANES):
        ones_v[pl.ds(k * LANES, LANES)] = jnp.ones((LANES,), jnp.float32)
    st = pl.ds(sid * DEG_STRIPE, DEG_STRIPE)
    pltpu.sync_copy(zdeg_h.at[st], dego_sh.at[st])
    pltpu.sync_copy(zdeg_h.at[st], degi_sh.at[st])
    plsc.subcore_barrier()
    GC = GRP * CHP
    base0 = wid * NCHP * CHP

    def group(gi, carry):
        gsl = pl.ds(base0 + gi * GC, GC)
        pltpu.sync_copy(src_h.at[gsl], src_g)
        pltpu.sync_copy(dst_h.at[gsl], dst_g)
        pltpu.sync_copy(et_h.at[gsl], et_g)

        @plsc.parallel_loop(0, GC, step=LANES, unroll=4)
        def compute(i):
            s = pl.ds(i, LANES)
            ev = et_g[s] * P
            ixs_g[s] = ev + src_g[s]
            ixd_g[s] = ev + dst_g[s]

        pltpu.sync_copy(ixs_g, idxs_out.at[gsl])
        pltpu.sync_copy(ixd_g, idxd_out.at[gsl])
        for k in range(GRP):
            b = k % 2
            if k >= 2:
                pltpu.make_async_copy(
                    ones_v, dego_sh.at[ico[b]], so[b]).wait()
                pltpu.make_async_copy(
                    ones_v, degi_sh.at[ici[b]], si[b]).wait()
            for q in range(CHP // LANES):
                s = pl.ds(q * LANES, LANES)
                ico[b][s] = ixs_g[pl.ds(k * CHP + q * LANES, LANES)]
                ici[b][s] = ixd_g[pl.ds(k * CHP + q * LANES, LANES)]
            pltpu.async_copy(ones_v, dego_sh.at[ico[b]], so[b], add=True)
            pltpu.async_copy(ones_v, degi_sh.at[ici[b]], si[b], add=True)
        for b in range(2):
            pltpu.make_async_copy(ones_v, dego_sh.at[ico[b]], so[b]).wait()
            pltpu.make_async_copy(ones_v, degi_sh.at[ici[b]], si[b]).wait()
        return carry

    lax.fori_loop(0, NCHP // GRP, group, 0)
    plsc.subcore_barrier()
    off = cid * 2 * T + sid * DEG_STRIPE
    pltpu.sync_copy(dego_sh.at[st], deg_out.at[pl.ds(off, DEG_STRIPE)])
    pltpu.sync_copy(degi_sh.at[st], deg_out.at[pl.ds(off + T, DEG_STRIPE)])


@functools.cache
def _sc_prep():
    return pl.kernel(
        _sc_prep_body,
        out_type=(
            jax.ShapeDtypeStruct((E_PAD,), jnp.int32),
            jax.ShapeDtypeStruct((E_PAD,), jnp.int32),
            jax.ShapeDtypeStruct((NC * 2 * T,), jnp.float32),
        ),
        mesh=_mesh(),
        compiler_params=pltpu.CompilerParams(needs_layout_passes=False),
        scratch_types=[
            pltpu.VMEM((GRP * CHP,), jnp.int32),
            pltpu.VMEM((GRP * CHP,), jnp.int32),
            pltpu.VMEM((GRP * CHP,), jnp.int32),
            pltpu.VMEM((GRP * CHP,), jnp.int32),
            pltpu.VMEM((GRP * CHP,), jnp.int32),
            pltpu.VMEM((CHP,), jnp.float32),
            pltpu.VMEM((CHP,), jnp.int32),
            pltpu.VMEM((CHP,), jnp.int32),
            pltpu.VMEM((CHP,), jnp.int32),
            pltpu.VMEM((CHP,), jnp.int32),
            pltpu.VMEM_SHARED((T,), jnp.float32),
            pltpu.VMEM_SHARED((T,), jnp.float32),
            pltpu.SemaphoreType.DMA,
            pltpu.SemaphoreType.DMA,
            pltpu.SemaphoreType.DMA,
            pltpu.SemaphoreType.DMA,
        ],
    )


def _tc_norm_body(deg_ref, no_ref, ni_ref):
    d = deg_ref[...]
    do = d[0, 0] + d[1, 0]
    di = d[0, 1] + d[1, 1]
    rowi = lax.broadcasted_iota(jnp.int32, (TROWS, 128), 0)
    coli = lax.broadcasted_iota(jnp.int32, (TROWS, 128), 1)
    mask = lax.rem(rowi * 128 + coli, P) < NUME
    no_ref[...] = jnp.where(mask, lax.rsqrt(jnp.maximum(do, 1.0)), 0.0)
    ni_ref[...] = jnp.where(mask, lax.rsqrt(jnp.maximum(di, 1.0)), 0.0)


def _tc_norm(degs4):
    return pl.pallas_call(
        _tc_norm_body,
        out_shape=(
            jax.ShapeDtypeStruct((TROWS, 128), jnp.float32),
            jax.ShapeDtypeStruct((TROWS, 128), jnp.float32),
        ),
    )(degs4)


def _sc_coef_body(normo_h, normi_h, idxs_h, idxd_h, c_out,
                  ixs, ixd, cg, a0, a1, b0, b1, sa0, sa1, sb0, sb1):
    wid = _wid()
    av = [a0, a1]
    bv = [b0, b1]
    sa = [sa0, sa1]
    sb = [sb0, sb1]
    GC = GRP * CHP
    base0 = wid * NCHP * CHP

    def group(gi, carry):
        gsl = pl.ds(base0 + gi * GC, GC)
        pltpu.sync_copy(idxs_h.at[gsl], ixs)
        pltpu.sync_copy(idxd_h.at[gsl], ixd)
        for p in range(2):
            pltpu.async_copy(
                normo_h.at[ixs.at[pl.ds(p * CHP, CHP)]], av[p], sa[p])
            pltpu.async_copy(
                normi_h.at[ixd.at[pl.ds(p * CHP, CHP)]], bv[p], sb[p])
        for k in range(GRP):
            b = k % 2
            ksl = pl.ds(k * CHP, CHP)
            pltpu.make_async_copy(
                normo_h.at[ixs.at[ksl]], av[b], sa[b]).wait()
            pltpu.make_async_copy(
                normi_h.at[ixd.at[ksl]], bv[b], sb[b]).wait()
            for q in range(CHP // LANES):
                s = pl.ds(q * LANES, LANES)
                cg[pl.ds(k * CHP + q * LANES, LANES)] = av[b][s] * bv[b][s]
            if k + 2 < GRP:
                nsl = pl.ds((k + 2) * CHP, CHP)
                pltpu.async_copy(normo_h.at[ixs.at[nsl]], av[b], sa[b])
                pltpu.async_copy(normi_h.at[ixd.at[nsl]], bv[b], sb[b])
        pltpu.sync_copy(cg, c_out.at[gsl])
        return carry

    lax.fori_loop(0, NCHP // GRP, group, 0)


@functools.cache
def _sc_coef():
    return pl.kernel(
        _sc_coef_body,
        out_type=jax.ShapeDtypeStruct((E_PAD,), jnp.float32),
        mesh=_mesh(),
        compiler_params=pltpu.CompilerParams(needs_layout_passes=False),
        scratch_types=[
            pltpu.VMEM((GRP * CHP,), jnp.int32),
            pltpu.VMEM((GRP * CHP,), jnp.int32),
            pltpu.VMEM((GRP * CHP,), jnp.float32),
            pltpu.VMEM((CHP,), jnp.float32),
            pltpu.VMEM((CHP,), jnp.float32),
            pltpu.VMEM((CHP,), jnp.float32),
            pltpu.VMEM((CHP,), jnp.float32),
            pltpu.SemaphoreType.DMA,
            pltpu.SemaphoreType.DMA,
            pltpu.SemaphoreType.DMA,
            pltpu.SemaphoreType.DMA,
        ],
    )


def _tc_g0_body(h_ref, w_ref, g_ref):
    g_ref[...] = jnp.dot(h_ref[...], w_ref[0],
                         preferred_element_type=jnp.float32)


def _tc_g0(h, w):
    return pl.pallas_call(
        _tc_g0_body,
        grid=(R, NT),
        in_specs=[
            pl.BlockSpec((M_T, DIM), lambda r, i: (i, 0)),
            pl.BlockSpec((1, DIM, DIM), lambda r, i: (r, 0, 0)),
        ],
        out_specs=pl.BlockSpec((M_T, DIM), lambda r, i: (r * NT + i, 0)),
        out_shape=jax.ShapeDtypeStruct((R * P, DIM), jnp.float32),
    )(h, w)


def _tc_g1_body(p0_ref, p1_ref, cb_ref, w_ref, g_ref):
    bm = jnp.sum(cb_ref[...], axis=0, keepdims=True) * (1.0 / R)
    h = (p0_ref[0] + p1_ref[0]) * (1.0 / R) + bm
    g_ref[...] = jnp.dot(h, w_ref[0], preferred_element_type=jnp.float32)


def _tc_g1(parts, cb, w):
    return pl.pallas_call(
        _tc_g1_body,
        grid=(R, NT),
        in_specs=[
            pl.BlockSpec((1, M_T, DIM), lambda r, i: (0, i, 0)),
            pl.BlockSpec((1, M_T, DIM), lambda r, i: (1, i, 0)),
            pl.BlockSpec((R, DIM), lambda r, i: (0, 0)),
            pl.BlockSpec((1, DIM, DIM), lambda r, i: (r, 0, 0)),
        ],
        out_specs=pl.BlockSpec((M_T, DIM), lambda r, i: (r * NT + i, 0)),
        out_shape=jax.ShapeDtypeStruct((R * P, DIM), jnp.float32),
    )(parts, parts, cb, w)


def _sc_scatter_body(g_h, idx_h, c_h, dst_h, zrow_h, part_out,
                     idx8, c8, dst8, dc0, dc1, rg0, rg1, rs0, rs1, accum_sh,
                     gs0, gs1, ss0, ss1):
    cid = lax.axis_index("c")
    sid = lax.axis_index("s")
    wid = _wid()
    st = pl.ds(sid * ACC_STRIPE, ACC_STRIPE)
    pltpu.sync_copy(zrow_h.at[st], accum_sh.at[st])
    plsc.subcore_barrier()
    dc = [dc0, dc1]
    rg = [rg0, rg1]
    rs = [rs0, rs1]
    gs = [gs0, gs1]
    ss = [ss0, ss1]
    col0 = lax.iota(jnp.int32, LANES)
    base0 = wid * NCH * CH

    def group(gi, carry):
        gbase = base0 + gi * GRP * CH
        gsl = pl.ds(gbase, GRP * CH)
        pltpu.sync_copy(idx_h.at[gsl], idx8)
        pltpu.sync_copy(c_h.at[gsl], c8)
        pltpu.sync_copy(dst_h.at[gsl], dst8)
        pltpu.async_copy(g_h.at[idx8.at[pl.ds(0, CH)]], rg[0], gs[0])
        pltpu.async_copy(g_h.at[idx8.at[pl.ds(CH, CH)]], rg[1], gs[1])
        for k in range(GRP):
            b = k % 2
            pltpu.make_async_copy(
                g_h.at[idx8.at[pl.ds(k * CH, CH)]], rg[b], gs[b]).wait()
            if k >= 2:
                pltpu.make_async_copy(rs[b], accum_sh.at[dc[b]], ss[b]).wait()
            for q in range(CH // LANES):
                dc[b][pl.ds(q * LANES, LANES)] = dst8[
                    pl.ds(k * CH + q * LANES, LANES)]
            kk = jnp.full((LANES,), k * CH, jnp.int32)

            @plsc.parallel_loop(0, CH, unroll=4)
            def scale(i, _b=b, _kk=kk):
                ii = jnp.zeros((LANES,), jnp.int32) + i
                cb = plsc.load_gather(c8, [_kk + ii])
                for q in range(DIM // LANES):
                    idxs = [ii, col0 + q * LANES]
                    v = plsc.load_gather(rg[_b], idxs)
                    plsc.store_scatter(rs[_b], idxs, v * cb)
            if k + 2 < GRP:
                pltpu.async_copy(
                    g_h.at[idx8.at[pl.ds((k + 2) * CH, CH)]], rg[b], gs[b])
            pltpu.async_copy(rs[b], accum_sh.at[dc[b]], ss[b], add=True)
        pltpu.make_async_copy(rs[0], accum_sh.at[dc[0]], ss[0]).wait()
        pltpu.make_async_copy(rs[1], accum_sh.at[dc[1]], ss[1]).wait()
        return carry

    lax.fori_loop(0, NGRP, group, 0)
    plsc.subcore_barrier()
    pltpu.sync_copy(accum_sh.at[st], part_out.at[cid, st])


@functools.cache
def _sc_scatter():
    return pl.kernel(
        _sc_scatter_body,
        out_type=jax.ShapeDtypeStruct((NC, P, DIM), jnp.float32),
        mesh=_mesh(),
        compiler_params=pltpu.CompilerParams(needs_layout_passes=False),
        scratch_types=[
            pltpu.VMEM((GRP * CH,), jnp.int32),
            pltpu.VMEM((GRP * CH,), jnp.float32),
            pltpu.VMEM((GRP * CH,), jnp.int32),
            pltpu.VMEM((CH,), jnp.int32),
            pltpu.VMEM((CH,), jnp.int32),
            pltpu.VMEM((CH, DIM), jnp.float32),
            pltpu.VMEM((CH, DIM), jnp.float32),
            pltpu.VMEM((CH, DIM), jnp.float32),
            pltpu.VMEM((CH, DIM), jnp.float32),
            pltpu.VMEM_SHARED((P, DIM), jnp.float32),
            pltpu.SemaphoreType.DMA,
            pltpu.SemaphoreType.DMA,
            pltpu.SemaphoreType.DMA,
            pltpu.SemaphoreType.DMA,
        ],
    )


def _tc_h_body(p0_ref, p1_ref, cb_ref, h_ref):
    bm = jnp.sum(cb_ref[...], axis=0, keepdims=True) * (1.0 / R)
    h_ref[...] = (p0_ref[0] + p1_ref[0]) * (1.0 / R) + bm


def _tc_h(parts, cb):
    return pl.pallas_call(
        _tc_h_body,
        grid=(NT,),
        in_specs=[
            pl.BlockSpec((1, M_T, DIM), lambda i: (0, i, 0)),
            pl.BlockSpec((1, M_T, DIM), lambda i: (1, i, 0)),
            pl.BlockSpec((R, DIM), lambda i: (0, 0)),
        ],
        out_specs=pl.BlockSpec((M_T, DIM), lambda i: (i, 0)),
        out_shape=jax.ShapeDtypeStruct((P, DIM), jnp.float32),
    )(parts, parts, cb)


def _sc_gather_body(h_h, sub_h, obj_h, rel_h, sre_h, ore_h,
                    hs_out, ho_out, sr_out, or_out,
                    i_v, hrows_v, rrows_v, sem):
    wid = _wid()
    sl = pl.ds(wid * BW, BW)
    pltpu.sync_copy(sub_h.at[sl], i_v)
    pltpu.async_copy(h_h.at[i_v], hrows_v, sem).wait()
    pltpu.sync_copy(hrows_v, hs_out.at[sl])
    pltpu.sync_copy(obj_h.at[sl], i_v)
    pltpu.async_copy(h_h.at[i_v], hrows_v, sem).wait()
    pltpu.sync_copy(hrows_v, ho_out.at[sl])
    pltpu.sync_copy(rel_h.at[sl], i_v)
    pltpu.async_copy(sre_h.at[i_v], rrows_v, sem).wait()
    pltpu.sync_copy(rrows_v, sr_out.at[sl])
    pltpu.async_copy(ore_h.at[i_v], rrows_v, sem).wait()
    pltpu.sync_copy(rrows_v, or_out.at[sl])


@functools.cache
def _sc_gather():
    return pl.kernel(
        _sc_gather_body,
        out_type=(
            jax.ShapeDtypeStruct((B, DIM), jnp.float32),
            jax.ShapeDtypeStruct((B, DIM), jnp.float32),
            jax.ShapeDtypeStruct((B, DIM), jnp.float32),
            jax.ShapeDtypeStruct((B, DIM), jnp.float32),
        ),
        mesh=_mesh(),
        compiler_params=pltpu.CompilerParams(needs_layout_passes=False),
        scratch_types=[
            pltpu.VMEM((BW,), jnp.int32),
            pltpu.VMEM((BW, DIM), jnp.float32),
            pltpu.VMEM((BW, DIM), jnp.float32),
            pltpu.SemaphoreType.DMA,
        ],
    )


def _tc_cls_body(ho_ref, or_ref, hs_ref, sr_ref,
                 swe_ref, swr_ref, sb_ref, owe_ref, owr_ref, ob_ref,
                 sp_ref, op_ref):
    sp_ref[...] = (jnp.dot(ho_ref[...], swe_ref[...],
                           preferred_element_type=jnp.float32)
                   + jnp.dot(or_ref[...], swr_ref[...],
                             preferred_element_type=jnp.float32)
                   + sb_ref[...])
    op_ref[...] = (jnp.dot(hs_ref[...], owe_ref[...],
                           preferred_element_type=jnp.float32)
                   + jnp.dot(sr_ref[...], owr_ref[...],
                             preferred_element_type=jnp.float32)
                   + ob_ref[...])


def _tc_cls(ho, orr, hs, sr, swe, swr, sb, owe, owr, ob):
    return pl.pallas_call(
        _tc_cls_body,
        grid=(N_PAD // N_T,),
        in_specs=[
            pl.BlockSpec((B, DIM), lambda n: (0, 0)),
            pl.BlockSpec((B, 32), lambda n: (0, 0)),
            pl.BlockSpec((B, DIM), lambda n: (0, 0)),
            pl.BlockSpec((B, 32), lambda n: (0, 0)),
            pl.BlockSpec((DIM, N_T), lambda n: (0, n)),
            pl.BlockSpec((32, N_T), lambda n: (0, n)),
            pl.BlockSpec((1, N_T), lambda n: (0, n)),
            pl.BlockSpec((DIM, N_T), lambda n: (0, n)),
            pl.BlockSpec((32, N_T), lambda n: (0, n)),
            pl.BlockSpec((1, N_T), lambda n: (0, n)),
        ],
        out_specs=(
            pl.BlockSpec((B, N_T), lambda n: (0, n)),
            pl.BlockSpec((B, N_T), lambda n: (0, n)),
        ),
        out_shape=(
            jax.ShapeDtypeStruct((B, N_PAD), jnp.float32),
            jax.ShapeDtypeStruct((B, N_PAD), jnp.float32),
        ),
    )(ho, orr, hs, sr, swe, swr, sb, owe, owr, ob)


def kernel(sub, obj, rel, edge_index, etype, ts, entity_emb, sub_rel_emb,
           obj_rel_emb, conv_W, conv_b, obj_cls_W, obj_cls_b, sub_cls_W,
           sub_cls_b):
    src = edge_index[0].astype(jnp.int32)
    dst = edge_index[1].astype(jnp.int32)
    et = etype.astype(jnp.int32)
    pad = E_PAD - E
    padv = NUME + (jnp.arange(pad, dtype=jnp.int32) % (P - NUME))
    src_p = jnp.concatenate([src, padv])
    dst_p = jnp.concatenate([dst, padv])
    et_p = jnp.concatenate([et, jnp.zeros((pad,), jnp.int32)])
    zdeg = jnp.zeros((T,), jnp.float32)
    zrow = jnp.zeros((P, DIM), jnp.float32)

    idx_s, idx_d, degs = _sc_prep()(src_p, dst_p, et_p, zdeg)
    normo, normi = _tc_norm(degs.reshape(NC, 2, TROWS, 128))
    c = _sc_coef()(normo.reshape(T), normi.reshape(T), idx_s, idx_d)

    emb_p = jnp.pad(entity_emb, ((0, P - NUME), (0, 0)))

    g0 = _tc_g0(emb_p, conv_W[0])
    parts0 = _sc_scatter()(g0, idx_s, c, dst_p, zrow)
    g1 = _tc_g1(parts0, conv_b[0], conv_W[1])
    parts1 = _sc_scatter()(g1, idx_s, c, dst_p, zrow)
    h2 = _tc_h(parts1, conv_b[1])

    srp = jnp.pad(sub_rel_emb, ((0, 0), (0, DIM - 32)))
    orp = jnp.pad(obj_rel_emb, ((0, 0), (0, DIM - 32)))
    hs, ho, sr, orr = _sc_gather()(h2, sub.astype(jnp.int32),
                                   obj.astype(jnp.int32),
                                   rel.astype(jnp.int32), srp, orp)
    sr = sr[:, :32]
    orr = orr[:, :32]

    npad = N_PAD - NUME
    swe = jnp.pad(sub_cls_W[:DIM], ((0, 0), (0, npad)))
    swr = jnp.pad(sub_cls_W[DIM:], ((0, 0), (0, npad)))
    sb = jnp.pad(sub_cls_b, (0, npad)).reshape(1, N_PAD)
    owe = jnp.pad(obj_cls_W[:DIM], ((0, 0), (0, npad)))
    owr = jnp.pad(obj_cls_W[DIM:], ((0, 0), (0, npad)))
    ob = jnp.pad(obj_cls_b, (0, npad)).reshape(1, N_PAD)
    sp, op_ = _tc_cls(ho, orr, hs, sr, swe, swr, sb, owe, owr, ob)
    return (sp[:, :NUME], op_[:, :NUME])


# speedup vs baseline: 72.8021x; 1.0400x over previous
import functools

import jax
import jax.numpy as jnp
from jax import lax
from jax.experimental import pallas as pl
from jax.experimental.pallas import tpu as pltpu
from jax.experimental.pallas import tpu_sc as plsc

NUME = 10000
DIM = 128
NUMR = 8
R = 2 * NUMR
B = 1024
E = 320000

NC = 2
NS = 16
LANES = 16
NW = NC * NS
CH = 80
GRP = 16
NCH = 128
NGRP = NCH // GRP
GRPP = 8
CHP = 128
NCHP = 80
E_PAD = NW * CH * NCH
P = 10240
T = R * P
TROWS = T // 128
DEG_STRIPE = T // NS
ACC_STRIPE = P // NS
M_T = 640
NT = P // M_T
N_PAD = 10240
N_T = 512
BW = B // NW


@functools.cache
def _mesh():
    return plsc.VectorSubcoreMesh(core_axis_name="c", subcore_axis_name="s")


def _wid():
    return lax.axis_index("s") * NC + lax.axis_index("c")


def _sc_prep_body(src_h, dst_h, et_h, zdeg_h, idxs_out, idxd_out, deg_out,
                  src_g, dst_g, et_g, ixs_g, ixd_g, ones_v,
                  ico0, ico1, ici0, ici1, dego_sh, degi_sh,
                  so0, so1, si0, si1):
    cid = lax.axis_index("c")
    sid = lax.axis_index("s")
    wid = _wid()
    ico = [ico0, ico1]
    ici = [ici0, ici1]
    so = [so0, so1]
    si = [si0, si1]
    for k in range(CHP // LANES):
        ones_v[pl.ds(k * LANES, LANES)] = jnp.ones((LANES,), jnp.float32)
    st = pl.ds(sid * DEG_STRIPE, DEG_STRIPE)
    pltpu.sync_copy(zdeg_h.at[st], dego_sh.at[st])
    pltpu.sync_copy(zdeg_h.at[st], degi_sh.at[st])
    plsc.subcore_barrier()
    GC = GRPP * CHP
    base0 = wid * NCHP * CHP

    def group(gi, carry):
        gsl = pl.ds(base0 + gi * GC, GC)
        pltpu.sync_copy(src_h.at[gsl], src_g)
        pltpu.sync_copy(dst_h.at[gsl], dst_g)
        pltpu.sync_copy(et_h.at[gsl], et_g)

        @plsc.parallel_loop(0, GC, step=LANES, unroll=4)
        def compute(i):
            s = pl.ds(i, LANES)
            ev = et_g[s] * P
            ixs_g[s] = ev + src_g[s]
            ixd_g[s] = ev + dst_g[s]

        pltpu.sync_copy(ixs_g, idxs_out.at[gsl])
        pltpu.sync_copy(ixd_g, idxd_out.at[gsl])
        for k in range(GRPP):
            b = k % 2
            if k >= 2:
                pltpu.make_async_copy(
                    ones_v, dego_sh.at[ico[b]], so[b]).wait()
                pltpu.make_async_copy(
                    ones_v, degi_sh.at[ici[b]], si[b]).wait()
            for q in range(CHP // LANES):
                s = pl.ds(q * LANES, LANES)
                ico[b][s] = ixs_g[pl.ds(k * CHP + q * LANES, LANES)]
                ici[b][s] = ixd_g[pl.ds(k * CHP + q * LANES, LANES)]
            pltpu.async_copy(ones_v, dego_sh.at[ico[b]], so[b], add=True)
            pltpu.async_copy(ones_v, degi_sh.at[ici[b]], si[b], add=True)
        for b in range(2):
            pltpu.make_async_copy(ones_v, dego_sh.at[ico[b]], so[b]).wait()
            pltpu.make_async_copy(ones_v, degi_sh.at[ici[b]], si[b]).wait()
        return carry

    lax.fori_loop(0, NCHP // GRPP, group, 0)
    plsc.subcore_barrier()
    off = cid * 2 * T + sid * DEG_STRIPE
    pltpu.sync_copy(dego_sh.at[st], deg_out.at[pl.ds(off, DEG_STRIPE)])
    pltpu.sync_copy(degi_sh.at[st], deg_out.at[pl.ds(off + T, DEG_STRIPE)])


@functools.cache
def _sc_prep():
    return pl.kernel(
        _sc_prep_body,
        out_type=(
            jax.ShapeDtypeStruct((E_PAD,), jnp.int32),
            jax.ShapeDtypeStruct((E_PAD,), jnp.int32),
            jax.ShapeDtypeStruct((NC * 2 * T,), jnp.float32),
        ),
        mesh=_mesh(),
        compiler_params=pltpu.CompilerParams(needs_layout_passes=False),
        scratch_types=[
            pltpu.VMEM((GRPP * CHP,), jnp.int32),
            pltpu.VMEM((GRPP * CHP,), jnp.int32),
            pltpu.VMEM((GRPP * CHP,), jnp.int32),
            pltpu.VMEM((GRPP * CHP,), jnp.int32),
            pltpu.VMEM((GRPP * CHP,), jnp.int32),
            pltpu.VMEM((CHP,), jnp.float32),
            pltpu.VMEM((CHP,), jnp.int32),
            pltpu.VMEM((CHP,), jnp.int32),
            pltpu.VMEM((CHP,), jnp.int32),
            pltpu.VMEM((CHP,), jnp.int32),
            pltpu.VMEM_SHARED((T,), jnp.float32),
            pltpu.VMEM_SHARED((T,), jnp.float32),
            pltpu.SemaphoreType.DMA,
            pltpu.SemaphoreType.DMA,
            pltpu.SemaphoreType.DMA,
            pltpu.SemaphoreType.DMA,
        ],
    )


def _tc_norm_body(deg_ref, no_ref, ni_ref):
    d = deg_ref[...]
    do = d[0, 0] + d[1, 0]
    di = d[0, 1] + d[1, 1]
    rowi = lax.broadcasted_iota(jnp.int32, (TROWS, 128), 0)
    coli = lax.broadcasted_iota(jnp.int32, (TROWS, 128), 1)
    mask = lax.rem(rowi * 128 + coli, P) < NUME
    no_ref[...] = jnp.where(mask, lax.rsqrt(jnp.maximum(do, 1.0)), 0.0)
    ni_ref[...] = jnp.where(mask, lax.rsqrt(jnp.maximum(di, 1.0)), 0.0)


def _tc_norm(degs4):
    return pl.pallas_call(
        _tc_norm_body,
        out_shape=(
            jax.ShapeDtypeStruct((TROWS, 128), jnp.float32),
            jax.ShapeDtypeStruct((TROWS, 128), jnp.float32),
        ),
    )(degs4)


def _sc_coef_body(normo_h, normi_h, idxs_h, idxd_h, c_out,
                  ixs, ixd, cg, a0, a1, b0, b1, sa0, sa1, sb0, sb1):
    wid = _wid()
    av = [a0, a1]
    bv = [b0, b1]
    sa = [sa0, sa1]
    sb = [sb0, sb1]
    GC = GRPP * CHP
    base0 = wid * NCHP * CHP

    def group(gi, carry):
        gsl = pl.ds(base0 + gi * GC, GC)
        pltpu.sync_copy(idxs_h.at[gsl], ixs)
        pltpu.sync_copy(idxd_h.at[gsl], ixd)
        for p in range(2):
            pltpu.async_copy(
                normo_h.at[ixs.at[pl.ds(p * CHP, CHP)]], av[p], sa[p])
            pltpu.async_copy(
                normi_h.at[ixd.at[pl.ds(p * CHP, CHP)]], bv[p], sb[p])
        for k in range(GRPP):
            b = k % 2
            ksl = pl.ds(k * CHP, CHP)
            pltpu.make_async_copy(
                normo_h.at[ixs.at[ksl]], av[b], sa[b]).wait()
            pltpu.make_async_copy(
                normi_h.at[ixd.at[ksl]], bv[b], sb[b]).wait()
            for q in range(CHP // LANES):
                s = pl.ds(q * LANES, LANES)
                cg[pl.ds(k * CHP + q * LANES, LANES)] = av[b][s] * bv[b][s]
            if k + 2 < GRPP:
                nsl = pl.ds((k + 2) * CHP, CHP)
                pltpu.async_copy(normo_h.at[ixs.at[nsl]], av[b], sa[b])
                pltpu.async_copy(normi_h.at[ixd.at[nsl]], bv[b], sb[b])
        pltpu.sync_copy(cg, c_out.at[gsl])
        return carry

    lax.fori_loop(0, NCHP // GRPP, group, 0)


@functools.cache
def _sc_coef():
    return pl.kernel(
        _sc_coef_body,
        out_type=jax.ShapeDtypeStruct((E_PAD,), jnp.float32),
        mesh=_mesh(),
        compiler_params=pltpu.CompilerParams(needs_layout_passes=False),
        scratch_types=[
            pltpu.VMEM((GRPP * CHP,), jnp.int32),
            pltpu.VMEM((GRPP * CHP,), jnp.int32),
            pltpu.VMEM((GRPP * CHP,), jnp.float32),
            pltpu.VMEM((CHP,), jnp.float32),
            pltpu.VMEM((CHP,), jnp.float32),
            pltpu.VMEM((CHP,), jnp.float32),
            pltpu.VMEM((CHP,), jnp.float32),
            pltpu.SemaphoreType.DMA,
            pltpu.SemaphoreType.DMA,
            pltpu.SemaphoreType.DMA,
            pltpu.SemaphoreType.DMA,
        ],
    )


def _tc_g0_body(h_ref, w_ref, g_ref):
    g_ref[...] = jnp.dot(h_ref[...], w_ref[0],
                         preferred_element_type=jnp.float32)


def _tc_g0(h, w):
    return pl.pallas_call(
        _tc_g0_body,
        grid=(R, NT),
        in_specs=[
            pl.BlockSpec((M_T, DIM), lambda r, i: (i, 0)),
            pl.BlockSpec((1, DIM, DIM), lambda r, i: (r, 0, 0)),
        ],
        out_specs=pl.BlockSpec((M_T, DIM), lambda r, i: (r * NT + i, 0)),
        out_shape=jax.ShapeDtypeStruct((R * P, DIM), jnp.float32),
    )(h, w)


def _tc_g1_body(p0_ref, p1_ref, cb_ref, w_ref, g_ref):
    bm = jnp.sum(cb_ref[...], axis=0, keepdims=True) * (1.0 / R)
    h = (p0_ref[0] + p1_ref[0]) * (1.0 / R) + bm
    g_ref[...] = jnp.dot(h, w_ref[0], preferred_element_type=jnp.float32)


def _tc_g1(parts, cb, w):
    return pl.pallas_call(
        _tc_g1_body,
        grid=(R, NT),
        in_specs=[
            pl.BlockSpec((1, M_T, DIM), lambda r, i: (0, i, 0)),
            pl.BlockSpec((1, M_T, DIM), lambda r, i: (1, i, 0)),
            pl.BlockSpec((R, DIM), lambda r, i: (0, 0)),
            pl.BlockSpec((1, DIM, DIM), lambda r, i: (r, 0, 0)),
        ],
        out_specs=pl.BlockSpec((M_T, DIM), lambda r, i: (r * NT + i, 0)),
        out_shape=jax.ShapeDtypeStruct((R * P, DIM), jnp.float32),
    )(parts, parts, cb, w)


def _sc_scatter_body(g_h, idx_h, c_h, dst_h, zrow_h, part_out,
                     idx8, c8, dst8, dc0, dc1, rg0, rg1, rs0, rs1, accum_sh,
                     gs0, gs1, ss0, ss1):
    cid = lax.axis_index("c")
    sid = lax.axis_index("s")
    wid = _wid()
    st = pl.ds(sid * ACC_STRIPE, ACC_STRIPE)
    pltpu.sync_copy(zrow_h.at[st], accum_sh.at[st])
    plsc.subcore_barrier()
    dc = [dc0, dc1]
    rg = [rg0, rg1]
    rs = [rs0, rs1]
    gs = [gs0, gs1]
    ss = [ss0, ss1]
    col0 = lax.iota(jnp.int32, LANES)
    base0 = wid * NCH * CH

    def group(gi, carry):
        gbase = base0 + gi * GRP * CH
        gsl = pl.ds(gbase, GRP * CH)
        pltpu.sync_copy(idx_h.at[gsl], idx8)
        pltpu.sync_copy(c_h.at[gsl], c8)
        pltpu.sync_copy(dst_h.at[gsl], dst8)
        pltpu.async_copy(g_h.at[idx8.at[pl.ds(0, CH)]], rg[0], gs[0])
        pltpu.async_copy(g_h.at[idx8.at[pl.ds(CH, CH)]], rg[1], gs[1])
        for k in range(GRP):
            b = k % 2
            pltpu.make_async_copy(
                g_h.at[idx8.at[pl.ds(k * CH, CH)]], rg[b], gs[b]).wait()
            if k >= 2:
                pltpu.make_async_copy(rs[b], accum_sh.at[dc[b]], ss[b]).wait()
            for q in range(CH // LANES):
                dc[b][pl.ds(q * LANES, LANES)] = dst8[
                    pl.ds(k * CH + q * LANES, LANES)]
            kk = jnp.full((LANES,), k * CH, jnp.int32)

            @plsc.parallel_loop(0, CH, unroll=4)
            def scale(i, _b=b, _kk=kk):
                ii = jnp.zeros((LANES,), jnp.int32) + i
                cb = plsc.load_gather(c8, [_kk + ii])
                for q in range(DIM // LANES):
                    idxs = [ii, col0 + q * LANES]
                    v = plsc.load_gather(rg[_b], idxs)
                    plsc.store_scatter(rs[_b], idxs, v * cb)
            if k + 2 < GRP:
                pltpu.async_copy(
                    g_h.at[idx8.at[pl.ds((k + 2) * CH, CH)]], rg[b], gs[b])
            pltpu.async_copy(rs[b], accum_sh.at[dc[b]], ss[b], add=True)
        pltpu.make_async_copy(rs[0], accum_sh.at[dc[0]], ss[0]).wait()
        pltpu.make_async_copy(rs[1], accum_sh.at[dc[1]], ss[1]).wait()
        return carry

    lax.fori_loop(0, NGRP, group, 0)
    plsc.subcore_barrier()
    pltpu.sync_copy(accum_sh.at[st], part_out.at[cid, st])


@functools.cache
def _sc_scatter():
    return pl.kernel(
        _sc_scatter_body,
        out_type=jax.ShapeDtypeStruct((NC, P, DIM), jnp.float32),
        mesh=_mesh(),
        compiler_params=pltpu.CompilerParams(needs_layout_passes=False),
        scratch_types=[
            pltpu.VMEM((GRP * CH,), jnp.int32),
            pltpu.VMEM((GRP * CH,), jnp.float32),
            pltpu.VMEM((GRP * CH,), jnp.int32),
            pltpu.VMEM((CH,), jnp.int32),
            pltpu.VMEM((CH,), jnp.int32),
            pltpu.VMEM((CH, DIM), jnp.float32),
            pltpu.VMEM((CH, DIM), jnp.float32),
            pltpu.VMEM((CH, DIM), jnp.float32),
            pltpu.VMEM((CH, DIM), jnp.float32),
            pltpu.VMEM_SHARED((P, DIM), jnp.float32),
            pltpu.SemaphoreType.DMA,
            pltpu.SemaphoreType.DMA,
            pltpu.SemaphoreType.DMA,
            pltpu.SemaphoreType.DMA,
        ],
    )


def _tc_h_body(p0_ref, p1_ref, cb_ref, h_ref):
    bm = jnp.sum(cb_ref[...], axis=0, keepdims=True) * (1.0 / R)
    h_ref[...] = (p0_ref[0] + p1_ref[0]) * (1.0 / R) + bm


def _tc_h(parts, cb):
    return pl.pallas_call(
        _tc_h_body,
        grid=(NT,),
        in_specs=[
            pl.BlockSpec((1, M_T, DIM), lambda i: (0, i, 0)),
            pl.BlockSpec((1, M_T, DIM), lambda i: (1, i, 0)),
            pl.BlockSpec((R, DIM), lambda i: (0, 0)),
        ],
        out_specs=pl.BlockSpec((M_T, DIM), lambda i: (i, 0)),
        out_shape=jax.ShapeDtypeStruct((P, DIM), jnp.float32),
    )(parts, parts, cb)


def _sc_gather_body(h_h, sub_h, obj_h, rel_h, sre_h, ore_h,
                    hs_out, ho_out, sr_out, or_out,
                    i_v, hrows_v, rrows_v, sem):
    wid = _wid()
    sl = pl.ds(wid * BW, BW)
    pltpu.sync_copy(sub_h.at[sl], i_v)
    pltpu.async_copy(h_h.at[i_v], hrows_v, sem).wait()
    pltpu.sync_copy(hrows_v, hs_out.at[sl])
    pltpu.sync_copy(obj_h.at[sl], i_v)
    pltpu.async_copy(h_h.at[i_v], hrows_v, sem).wait()
    pltpu.sync_copy(hrows_v, ho_out.at[sl])
    pltpu.sync_copy(rel_h.at[sl], i_v)
    pltpu.async_copy(sre_h.at[i_v], rrows_v, sem).wait()
    pltpu.sync_copy(rrows_v, sr_out.at[sl])
    pltpu.async_copy(ore_h.at[i_v], rrows_v, sem).wait()
    pltpu.sync_copy(rrows_v, or_out.at[sl])


@functools.cache
def _sc_gather():
    return pl.kernel(
        _sc_gather_body,
        out_type=(
            jax.ShapeDtypeStruct((B, DIM), jnp.float32),
            jax.ShapeDtypeStruct((B, DIM), jnp.float32),
            jax.ShapeDtypeStruct((B, DIM), jnp.float32),
            jax.ShapeDtypeStruct((B, DIM), jnp.float32),
        ),
        mesh=_mesh(),
        compiler_params=pltpu.CompilerParams(needs_layout_passes=False),
        scratch_types=[
            pltpu.VMEM((BW,), jnp.int32),
            pltpu.VMEM((BW, DIM), jnp.float32),
            pltpu.VMEM((BW, DIM), jnp.float32),
            pltpu.SemaphoreType.DMA,
        ],
    )


def _tc_cls_body(ho_ref, or_ref, hs_ref, sr_ref,
                 swe_ref, swr_ref, sb_ref, owe_ref, owr_ref, ob_ref,
                 sp_ref, op_ref):
    sp_ref[...] = (jnp.dot(ho_ref[...], swe_ref[...],
                           preferred_element_type=jnp.float32)
                   + jnp.dot(or_ref[...], swr_ref[...],
                             preferred_element_type=jnp.float32)
                   + sb_ref[...])
    op_ref[...] = (jnp.dot(hs_ref[...], owe_ref[...],
                           preferred_element_type=jnp.float32)
                   + jnp.dot(sr_ref[...], owr_ref[...],
                             preferred_element_type=jnp.float32)
                   + ob_ref[...])


def _tc_cls(ho, orr, hs, sr, swe, swr, sb, owe, owr, ob):
    return pl.pallas_call(
        _tc_cls_body,
        grid=(N_PAD // N_T,),
        in_specs=[
            pl.BlockSpec((B, DIM), lambda n: (0, 0)),
            pl.BlockSpec((B, 32), lambda n: (0, 0)),
            pl.BlockSpec((B, DIM), lambda n: (0, 0)),
            pl.BlockSpec((B, 32), lambda n: (0, 0)),
            pl.BlockSpec((DIM, N_T), lambda n: (0, n)),
            pl.BlockSpec((32, N_T), lambda n: (0, n)),
            pl.BlockSpec((1, N_T), lambda n: (0, n)),
            pl.BlockSpec((DIM, N_T), lambda n: (0, n)),
            pl.BlockSpec((32, N_T), lambda n: (0, n)),
            pl.BlockSpec((1, N_T), lambda n: (0, n)),
        ],
        out_specs=(
            pl.BlockSpec((B, N_T), lambda n: (0, n)),
            pl.BlockSpec((B, N_T), lambda n: (0, n)),
        ),
        out_shape=(
            jax.ShapeDtypeStruct((B, N_PAD), jnp.float32),
            jax.ShapeDtypeStruct((B, N_PAD), jnp.float32),
        ),
    )(ho, orr, hs, sr, swe, swr, sb, owe, owr, ob)


def kernel(sub, obj, rel, edge_index, etype, ts, entity_emb, sub_rel_emb,
           obj_rel_emb, conv_W, conv_b, obj_cls_W, obj_cls_b, sub_cls_W,
           sub_cls_b):
    src = edge_index[0].astype(jnp.int32)
    dst = edge_index[1].astype(jnp.int32)
    et = etype.astype(jnp.int32)
    pad = E_PAD - E
    padv = NUME + (jnp.arange(pad, dtype=jnp.int32) % (P - NUME))
    src_p = jnp.concatenate([src, padv])
    dst_p = jnp.concatenate([dst, padv])
    et_p = jnp.concatenate([et, jnp.zeros((pad,), jnp.int32)])
    zdeg = jnp.zeros((T,), jnp.float32)
    zrow = jnp.zeros((P, DIM), jnp.float32)

    idx_s, idx_d, degs = _sc_prep()(src_p, dst_p, et_p, zdeg)
    normo, normi = _tc_norm(degs.reshape(NC, 2, TROWS, 128))
    c = _sc_coef()(normo.reshape(T), normi.reshape(T), idx_s, idx_d)

    emb_p = jnp.pad(entity_emb, ((0, P - NUME), (0, 0)))

    g0 = _tc_g0(emb_p, conv_W[0])
    parts0 = _sc_scatter()(g0, idx_s, c, dst_p, zrow)
    g1 = _tc_g1(parts0, conv_b[0], conv_W[1])
    parts1 = _sc_scatter()(g1, idx_s, c, dst_p, zrow)
    h2 = _tc_h(parts1, conv_b[1])

    srp = jnp.pad(sub_rel_emb, ((0, 0), (0, DIM - 32)))
    orp = jnp.pad(obj_rel_emb, ((0, 0), (0, DIM - 32)))
    hs, ho, sr, orr = _sc_gather()(h2, sub.astype(jnp.int32),
                                   obj.astype(jnp.int32),
                                   rel.astype(jnp.int32), srp, orp)
    sr = sr[:, :32]
    orr = orr[:, :32]

    npad = N_PAD - NUME
    swe = jnp.pad(sub_cls_W[:DIM], ((0, 0), (0, npad)))
    swr = jnp.pad(sub_cls_W[DIM:], ((0, 0), (0, npad)))
    sb = jnp.pad(sub_cls_b, (0, npad)).reshape(1, N_PAD)
    owe = jnp.pad(obj_cls_W[:DIM], ((0, 0), (0, npad)))
    owr = jnp.pad(obj_cls_W[DIM:], ((0, 0), (0, npad)))
    ob = jnp.pad(obj_cls_b, (0, npad)).reshape(1, N_PAD)
    sp, op_ = _tc_cls(ho, orr, hs, sr, swe, swr, sb, owe, owr, ob)
    return (sp[:, :NUME], op_[:, :NUME])
